# Initial kernel scaffold; baseline (speedup 1.0000x reference)
import functools

import jax
import jax.numpy as jnp
import numpy as np
from jax import lax
from jax.experimental import pallas as pl
from jax.experimental.pallas import tpu as pltpu
from jax.experimental.pallas import tpu_sc as plsc

N_NODES = 100000
N_EDGES = 1600000
NUM_GRAPHS = 64
ATOM_DIM = 13
TIME_DIM = 10
FEATS = ATOM_DIM + TIME_DIM
POS = 3
M_DIM = 16
NUM_LAYERS = 3

XD = 32
H1 = 96
NH = 48
CW = 64

NC, NS = 2, 16
NW = NC * NS
LANE = 128
EP = 1638400
IDX_ROWS = EP // LANE
ROWS_PER_W = IDX_ROWS // NW
CR = 8
G_CHUNKS = ROWS_PER_W // CR
ROWS_PER_T = IDX_ROWS // NS
S_CHUNKS = ROWS_PER_T // CR
NSTRIPE = N_NODES // NS

BE = 2048
BN = 2000


def _silu(x):
    return x * jax.nn.sigmoid(x)


def _sc_gather_body(x_hbm, srcp_hbm, dstp_hbm, gs_hbm, gd_hbm,
                    idx_s, idx_d, rows_s, rows_d, sem):
    cid = lax.axis_index("c")
    sid = lax.axis_index("s")
    wid = sid * NC + cid

    def chunk(c, carry):
        row0 = wid * ROWS_PER_W + c * CR
        pltpu.sync_copy(srcp_hbm.at[pl.ds(row0, CR)], idx_s)
        pltpu.sync_copy(dstp_hbm.at[pl.ds(row0, CR)], idx_d)
        cps = []
        for j in range(CR):
            cps.append(pltpu.async_copy(
                x_hbm.at[idx_s.at[j]], rows_s.at[pl.ds(j * LANE, LANE)], sem))
            cps.append(pltpu.async_copy(
                x_hbm.at[idx_d.at[j]], rows_d.at[pl.ds(j * LANE, LANE)], sem))
        for cp in cps:
            cp.wait()
        e0 = row0 * LANE
        pltpu.sync_copy(rows_s, gs_hbm.at[pl.ds(e0, CR * LANE)])
        pltpu.sync_copy(rows_d, gd_hbm.at[pl.ds(e0, CR * LANE)])
        return carry

    lax.fori_loop(0, G_CHUNKS, chunk, 0)


def _sc_gather(x, srcp, dstp):
    mesh = plsc.VectorSubcoreMesh(core_axis_name="c", subcore_axis_name="s")
    f = pl.kernel(
        _sc_gather_body,
        out_type=(jax.ShapeDtypeStruct((EP, XD), jnp.float32),
                  jax.ShapeDtypeStruct((EP, XD), jnp.float32)),
        mesh=mesh,
        scratch_types=[
            pltpu.VMEM((CR, LANE), jnp.int32),
            pltpu.VMEM((CR, LANE), jnp.int32),
            pltpu.VMEM((CR * LANE, XD), jnp.float32),
            pltpu.VMEM((CR * LANE, XD), jnp.float32),
            pltpu.SemaphoreType.DMA,
        ],
    )
    return f(x, srcp, dstp)


def _sc_scatter_body(outm_hbm, outw_hbm, dstp_hbm, zeros_hbm,
                     accm_hbm, accw_hbm, idx, upd, acc_shared, sem):
    cid = lax.axis_index("c")
    sid = lax.axis_index("s")

    pltpu.sync_copy(zeros_hbm.at[pl.ds(sid * NSTRIPE, NSTRIPE)],
                    acc_shared.at[pl.ds(sid * NSTRIPE, NSTRIPE)])
    plsc.subcore_barrier()

    def run(in_hbm):
        def chunk(c, carry):
            row0 = sid * ROWS_PER_T + c * CR
            pltpu.sync_copy(dstp_hbm.at[pl.ds(row0, CR)], idx)
            pltpu.sync_copy(in_hbm.at[pl.ds(row0 * LANE, CR * LANE)], upd)
            for j in range(CR):
                pltpu.sync_copy(upd.at[pl.ds(j * LANE, LANE)],
                                acc_shared.at[idx.at[j]], add=True)
            return carry
        lax.fori_loop(0, S_CHUNKS, chunk, 0)

    @pl.when(cid == 0)
    def _():
        run(outm_hbm)

    @pl.when(cid == 1)
    def _():
        run(outw_hbm)

    plsc.subcore_barrier()

    @pl.when(cid == 0)
    def _():
        pltpu.sync_copy(acc_shared.at[pl.ds(sid * NSTRIPE, NSTRIPE)],
                        accm_hbm.at[pl.ds(sid * NSTRIPE, NSTRIPE)])

    @pl.when(cid == 1)
    def _():
        pltpu.sync_copy(acc_shared.at[pl.ds(sid * NSTRIPE, NSTRIPE)],
                        accw_hbm.at[pl.ds(sid * NSTRIPE, NSTRIPE)])


def _sc_scatter(outm, outw, dstp, zeros_n16):
    mesh = plsc.VectorSubcoreMesh(core_axis_name="c", subcore_axis_name="s")
    f = pl.kernel(
        _sc_scatter_body,
        out_type=(jax.ShapeDtypeStruct((N_NODES, M_DIM), jnp.float32),
                  jax.ShapeDtypeStruct((N_NODES, M_DIM), jnp.float32)),
        mesh=mesh,
        scratch_types=[
            pltpu.VMEM((CR, LANE), jnp.int32),
            pltpu.VMEM((CR * LANE, M_DIM), jnp.float32),
            pltpu.VMEM_SHARED((N_NODES, M_DIM), jnp.float32),
            pltpu.SemaphoreType.DMA,
        ],
    )
    return f(outm, outw, dstp, zeros_n16)


def _tc_edge_body(gd_ref, gs_ref, a1_ref, b1_ref, w1d_ref, b1b_ref,
                  w2_ref, b2_ref, cw1_ref, cb1_ref, cw2_ref, cb2_ref,
                  outm_ref, outw_ref):
    pid = pl.program_id(0)
    gd = gd_ref[...]
    gs = gs_ref[...]
    mask3 = (lax.broadcasted_iota(jnp.int32, (1, XD), 1) < POS
             ).astype(jnp.float32)
    rel = (gs - gd) * mask3
    rel_dist = jnp.sum(rel * rel, axis=1, keepdims=True)
    h1 = (jnp.dot(gd, a1_ref[...], preferred_element_type=jnp.float32)
          + jnp.dot(gs, b1_ref[...], preferred_element_type=jnp.float32)
          + rel_dist * w1d_ref[...] + b1b_ref[...])
    a1 = _silu(h1)
    m = _silu(jnp.dot(a1, w2_ref[...], preferred_element_type=jnp.float32)
              + b2_ref[...])
    c1 = _silu(jnp.dot(m, cw1_ref[...], preferred_element_type=jnp.float32)
               + cb1_ref[...])
    cw = (jnp.dot(c1, cw2_ref[...], preferred_element_type=jnp.float32)
          + cb2_ref[...])
    cw0 = cw[:, 0:1]
    rows = lax.broadcasted_iota(jnp.int32, (BE, 1), 0) + pid * BE
    valid = (rows < N_EDGES).astype(jnp.float32)
    outm_ref[...] = m * valid
    outw_ref[...] = (rel * (cw0 * valid))[:, 0:M_DIM]


def _tc_edge(gd, gs, wts):
    (a1, b1, w1d, b1b, w2, b2, cw1, cb1, cw2, cb2) = wts
    nblk = EP // BE
    full = lambda shape: pl.BlockSpec(shape, lambda i: (0,) * len(shape))
    return pl.pallas_call(
        _tc_edge_body,
        grid=(nblk,),
        in_specs=[
            pl.BlockSpec((BE, XD), lambda i: (i, 0)),
            pl.BlockSpec((BE, XD), lambda i: (i, 0)),
            full((XD, H1)), full((XD, H1)), full((1, H1)), full((1, H1)),
            full((H1, M_DIM)), full((1, M_DIM)),
            full((M_DIM, CW)), full((1, CW)),
            full((CW, 8)), full((1, 8)),
        ],
        out_specs=[
            pl.BlockSpec((BE, M_DIM), lambda i: (i, 0)),
            pl.BlockSpec((BE, M_DIM), lambda i: (i, 0)),
        ],
        out_shape=[
            jax.ShapeDtypeStruct((EP, M_DIM), jnp.float32),
            jax.ShapeDtypeStruct((EP, M_DIM), jnp.float32),
        ],
        compiler_params=pltpu.CompilerParams(
            dimension_semantics=("arbitrary",)),
    )(gd, gs, a1, b1, w1d, b1b, w2, b2, cw1, cb1, cw2, cb2)


def _tc_node_body(x_ref, accm_ref, accw_ref, x1_ref, m1_ref, nb1_ref,
                  nw2_ref, nb2_ref, out_ref):
    x = x_ref[...]
    accm = accm_ref[...]
    accw = accw_ref[...]
    h = _silu(jnp.dot(x, x1_ref[...], preferred_element_type=jnp.float32)
              + jnp.dot(accm, m1_ref[...], preferred_element_type=jnp.float32)
              + nb1_ref[...])
    dx = jnp.dot(h, nw2_ref[...], preferred_element_type=jnp.float32) \
        + nb2_ref[...]
    mask3 = (lax.broadcasted_iota(jnp.int32, (1, XD), 1) < POS
             ).astype(jnp.float32)
    mhat = jnp.concatenate(
        [accw, jnp.zeros((BN, XD - M_DIM), jnp.float32)], axis=1) * mask3
    out_ref[...] = x + dx + mhat


def _tc_node(x, accm, accw, wts):
    (x1, m1, nb1, nw2, nb2) = wts
    nblk = N_NODES // BN
    full = lambda shape: pl.BlockSpec(shape, lambda i: (0,) * len(shape))
    return pl.pallas_call(
        _tc_node_body,
        grid=(nblk,),
        in_specs=[
            pl.BlockSpec((BN, XD), lambda i: (i, 0)),
            pl.BlockSpec((BN, M_DIM), lambda i: (i, 0)),
            pl.BlockSpec((BN, M_DIM), lambda i: (i, 0)),
            full((XD, NH)), full((M_DIM, NH)), full((1, NH)),
            full((NH, XD)), full((1, XD)),
        ],
        out_specs=pl.BlockSpec((BN, XD), lambda i: (i, 0)),
        out_shape=jax.ShapeDtypeStruct((N_NODES, XD), jnp.float32),
        compiler_params=pltpu.CompilerParams(
            dimension_semantics=("arbitrary",)),
    )(x, accm, accw, x1, m1, nb1, nw2, nb2)


def _tc_temb_body(t_ref, w1_ref, b1_ref, w2_ref, b2_ref, out_ref):
    half = TIME_DIM // 2
    freqs = jnp.asarray(
        np.exp(np.arange(half, dtype=np.float32)
               * (-(np.log(10000.0) / (half - 1)))).reshape(1, half))
    emb5 = t_ref[...] * freqs
    emb = jnp.concatenate([jnp.sin(emb5), jnp.cos(emb5)], axis=1)
    h = jnp.dot(emb, w1_ref[...], preferred_element_type=jnp.float32) \
        + b1_ref[...]
    h = 0.5 * h * (1.0 + lax.erf(h / np.float32(np.sqrt(2.0))))
    out_ref[...] = jnp.dot(h, w2_ref[...],
                           preferred_element_type=jnp.float32) + b2_ref[...]


def _tc_temb(t2, w1, b1, w2, b2):
    return pl.pallas_call(
        _tc_temb_body,
        out_shape=jax.ShapeDtypeStruct((NUM_GRAPHS, TIME_DIM), jnp.float32),
    )(t2, w1, b1, w2, b2)


def _tc_prep_body(pos_ref, v_ref, b_ref, temb_ref, out_ref):
    b = b_ref[0, 0, :].reshape(BN, 1)
    gids = lax.broadcasted_iota(jnp.int32, (1, NUM_GRAPHS), 1)
    oh = (b == gids).astype(jnp.float32)
    te = jnp.dot(oh, temb_ref[...], preferred_element_type=jnp.float32)
    base = jnp.concatenate(
        [pos_ref[...], v_ref[...],
         jnp.zeros((BN, XD - POS - ATOM_DIM), jnp.float32)], axis=1)
    out_ref[...] = base + te


def _tc_prep(pos, v, batchp, tembp):
    nblk = N_NODES // BN
    full = lambda shape: pl.BlockSpec(shape, lambda i: (0,) * len(shape))
    return pl.pallas_call(
        _tc_prep_body,
        grid=(nblk,),
        in_specs=[
            pl.BlockSpec((BN, POS), lambda i: (i, 0)),
            pl.BlockSpec((BN, ATOM_DIM), lambda i: (i, 0)),
            pl.BlockSpec((1, 1, BN), lambda i: (i, 0, 0)),
            full((NUM_GRAPHS, XD)),
        ],
        out_specs=pl.BlockSpec((BN, XD), lambda i: (i, 0)),
        out_shape=jax.ShapeDtypeStruct((N_NODES, XD), jnp.float32),
        compiler_params=pltpu.CompilerParams(
            dimension_semantics=("arbitrary",)),
    )(pos, v, batchp, tembp)


def _tc_pool_body(x_ref, b_ref, sh_ref, c24_ref, s_ref):
    pid = pl.program_id(0)
    b = b_ref[0, 0, :].reshape(BN, 1)
    gids = lax.broadcasted_iota(jnp.int32, (1, NUM_GRAPHS), 1)
    oh = (b == gids).astype(jnp.float32)
    y = jnp.dot(x_ref[...], sh_ref[...],
                preferred_element_type=jnp.float32) + c24_ref[...]
    part = lax.dot_general(oh, y, (((0,), (0,)), ((), ())),
                           preferred_element_type=jnp.float32)

    @pl.when(pid == 0)
    def _():
        s_ref[...] = part

    @pl.when(pid != 0)
    def _():
        s_ref[...] = s_ref[...] + part


def _tc_pool(x, batchp, sh, c24):
    nblk = N_NODES // BN
    full = lambda shape: pl.BlockSpec(shape, lambda i: (0,) * len(shape))
    return pl.pallas_call(
        _tc_pool_body,
        grid=(nblk,),
        in_specs=[
            pl.BlockSpec((BN, XD), lambda i: (i, 0)),
            pl.BlockSpec((1, 1, BN), lambda i: (i, 0, 0)),
            full((XD, XD)), full((1, XD)),
        ],
        out_specs=pl.BlockSpec((NUM_GRAPHS, XD), lambda i: (0, 0)),
        out_shape=jax.ShapeDtypeStruct((NUM_GRAPHS, XD), jnp.float32),
        compiler_params=pltpu.CompilerParams(
            dimension_semantics=("arbitrary",)),
    )(x, batchp, sh, c24)


def _tc_head_body(s_ref, w1_ref, b1_ref, w2_ref, b2_ref, out_ref):
    s = s_ref[...]
    cnt = jnp.maximum(s[:, FEATS:FEATS + 1], 1.0)
    p = s / cnt
    h = jnp.maximum(
        jnp.dot(p, w1_ref[...], preferred_element_type=jnp.float32)
        + b1_ref[...], 0.0)
    out_ref[...] = jnp.dot(h, w2_ref[...],
                           preferred_element_type=jnp.float32) + b2_ref[...]


def _tc_head(s, w1, b1, w2, b2):
    return pl.pallas_call(
        _tc_head_body,
        out_shape=jax.ShapeDtypeStruct((NUM_GRAPHS, 8), jnp.float32),
    )(s, w1, b1, w2, b2)


def _pack_layer(p):
    e_w1, e_b1 = p["e_w1"], p["e_b1"]
    e_w2, e_b2 = p["e_w2"], p["e_b2"]
    c_w1, c_b1 = p["c_w1"], p["c_b1"]
    c_w2, c_b2 = p["c_w2"], p["c_b2"]
    n_w1, n_b1 = p["n_w1"], p["n_b1"]
    n_w2, n_b2 = p["n_w2"], p["n_b2"]

    a1 = jnp.zeros((XD, H1), jnp.float32).at[POS:POS + FEATS, :94].set(
        e_w1[:FEATS])
    b1 = jnp.zeros((XD, H1), jnp.float32).at[POS:POS + FEATS, :94].set(
        e_w1[FEATS:2 * FEATS])
    w1d = jnp.zeros((1, H1), jnp.float32).at[0, :94].set(e_w1[2 * FEATS])
    b1b = jnp.zeros((1, H1), jnp.float32).at[0, :94].set(e_b1)
    w2 = jnp.zeros((H1, M_DIM), jnp.float32).at[:94].set(e_w2)
    b2 = e_b2.reshape(1, M_DIM)
    cw1 = c_w1
    cb1 = c_b1.reshape(1, CW)
    cw2 = jnp.zeros((CW, 8), jnp.float32).at[:, 0:1].set(c_w2)
    cb2 = jnp.zeros((1, 8), jnp.float32).at[0, 0].set(c_b2[0])
    ew = (a1, b1, w1d, b1b, w2, b2, cw1, cb1, cw2, cb2)

    x1 = jnp.zeros((XD, NH), jnp.float32).at[POS:POS + FEATS, :46].set(
        n_w1[:FEATS])
    m1 = jnp.zeros((M_DIM, NH), jnp.float32).at[:, :46].set(n_w1[FEATS:])
    nb1 = jnp.zeros((1, NH), jnp.float32).at[0, :46].set(n_b1)
    nw2 = jnp.zeros((NH, XD), jnp.float32).at[:46, POS:POS + FEATS].set(n_w2)
    nb2 = jnp.zeros((1, XD), jnp.float32).at[0, POS:POS + FEATS].set(n_b2)
    nw = (x1, m1, nb1, nw2, nb2)
    return ew, nw


def kernel(ligand_pos, ligand_v, edge_index, t, batch, params):
    npad = EP - N_EDGES
    pad_idx = (jnp.arange(npad, dtype=jnp.int32) * 37) % N_NODES
    srcp = jnp.concatenate([edge_index[0], pad_idx]).reshape(IDX_ROWS, LANE)
    dstp = jnp.concatenate([edge_index[1], pad_idx]).reshape(IDX_ROWS, LANE)
    batchp = batch.reshape(N_NODES // BN, 1, BN)
    zeros_n16 = jnp.zeros((N_NODES, M_DIM), jnp.float32)

    temb = _tc_temb(t.reshape(NUM_GRAPHS, 1),
                    params["te_w1"], params["te_b1"].reshape(1, -1),
                    params["te_w2"], params["te_b2"].reshape(1, -1))
    tembp = jnp.zeros((NUM_GRAPHS, XD), jnp.float32
                      ).at[:, POS + ATOM_DIM:POS + ATOM_DIM + TIME_DIM].set(
                          temb)
    x = _tc_prep(ligand_pos, ligand_v, batchp, tembp)

    for l in range(NUM_LAYERS):
        ew, nw = _pack_layer(params["layers"][l])
        gs, gd = _sc_gather(x, srcp, dstp)
        outm, outw = _tc_edge(gd, gs, ew)
        accm, accw = _sc_scatter(outm, outw, dstp, zeros_n16)
        x = _tc_node(x, accm, accw, nw)

    sh = jnp.zeros((XD, XD), jnp.float32).at[
        POS:POS + FEATS, 0:FEATS].set(jnp.eye(FEATS, dtype=jnp.float32))
    c24 = jnp.zeros((1, XD), jnp.float32).at[0, FEATS].set(1.0)
    s = _tc_pool(x, batchp, sh, c24)
    hw1 = jnp.zeros((XD, NH), jnp.float32).at[:FEATS, :46].set(params["d_w1"])
    hb1 = jnp.zeros((1, NH), jnp.float32).at[0, :46].set(params["d_b1"])
    hw2 = jnp.zeros((NH, 8), jnp.float32).at[:46, :2].set(params["d_w2"])
    hb2 = jnp.zeros((1, 8), jnp.float32).at[0, :2].set(params["d_b2"])
    out8 = _tc_head(s, hw1, hb1, hw2, hb2)
    return out8[:, :2]


# speedup vs baseline: 4.0386x; 4.0386x over previous
import functools

import jax
import jax.numpy as jnp
import numpy as np
from jax import lax
from jax.experimental import pallas as pl
from jax.experimental.pallas import tpu as pltpu
from jax.experimental.pallas import tpu_sc as plsc

N_NODES = 100000
N_EDGES = 1600000
NUM_GRAPHS = 64
ATOM_DIM = 13
TIME_DIM = 10
FEATS = ATOM_DIM + TIME_DIM
POS = 3
M_DIM = 16
NUM_LAYERS = 3

XD = 32
H1 = 96
NH = 48
CW = 64

NC, NS = 2, 16
NW = NC * NS
LANE = 128
EP = 1638400
IDX_ROWS = EP // LANE
ROWS_PER_W = IDX_ROWS // NW
CR = 8
G_CHUNKS = ROWS_PER_W // CR
ROWS_PER_T = IDX_ROWS // NS
S_CHUNKS = ROWS_PER_T // CR
NSTRIPE = N_NODES // NS

BE = 2048
BN = 2000


def _silu(x):
    return x * jax.nn.sigmoid(x)


def _sc_gather_body(x_hbm, srcp_hbm, dstp_hbm, gs_hbm, gd_hbm,
                    idx_s, idx_d, rows_s, rows_d, sem):
    cid = lax.axis_index("c")
    sid = lax.axis_index("s")
    wid = sid * NC + cid

    def chunk(c, carry):
        row0 = wid * ROWS_PER_W + c * CR
        pltpu.sync_copy(srcp_hbm.at[pl.ds(row0, CR)], idx_s)
        pltpu.sync_copy(dstp_hbm.at[pl.ds(row0, CR)], idx_d)
        cps = []
        for j in range(CR):
            cps.append(pltpu.async_copy(
                x_hbm.at[idx_s.at[j]], rows_s.at[pl.ds(j * LANE, LANE)], sem))
            cps.append(pltpu.async_copy(
                x_hbm.at[idx_d.at[j]], rows_d.at[pl.ds(j * LANE, LANE)], sem))
        for cp in cps:
            cp.wait()
        e0 = row0 * LANE
        pltpu.sync_copy(rows_s, gs_hbm.at[pl.ds(e0, CR * LANE)])
        pltpu.sync_copy(rows_d, gd_hbm.at[pl.ds(e0, CR * LANE)])
        return carry

    lax.fori_loop(0, G_CHUNKS, chunk, 0)


def _sc_gather(x, srcp, dstp):
    mesh = plsc.VectorSubcoreMesh(core_axis_name="c", subcore_axis_name="s")
    f = pl.kernel(
        _sc_gather_body,
        out_type=(jax.ShapeDtypeStruct((EP, XD), jnp.float32),
                  jax.ShapeDtypeStruct((EP, XD), jnp.float32)),
        mesh=mesh,
        scratch_types=[
            pltpu.VMEM((CR, LANE), jnp.int32),
            pltpu.VMEM((CR, LANE), jnp.int32),
            pltpu.VMEM((CR * LANE, XD), jnp.float32),
            pltpu.VMEM((CR * LANE, XD), jnp.float32),
            pltpu.SemaphoreType.DMA,
        ],
        compiler_params=pltpu.CompilerParams(use_tc_tiling_on_sc=False),
    )
    return f(x, srcp, dstp)


def _sc_scatter_body(outm_hbm, outw_hbm, dstp_hbm, zeros_hbm,
                     accm_hbm, accw_hbm, idx, upd, acc_shared, sem):
    cid = lax.axis_index("c")
    sid = lax.axis_index("s")

    pltpu.sync_copy(zeros_hbm.at[pl.ds(sid * NSTRIPE, NSTRIPE)],
                    acc_shared.at[pl.ds(sid * NSTRIPE, NSTRIPE)])
    plsc.subcore_barrier()

    def run(in_hbm):
        def chunk(c, carry):
            row0 = sid * ROWS_PER_T + c * CR
            pltpu.sync_copy(dstp_hbm.at[pl.ds(row0, CR)], idx)
            pltpu.sync_copy(in_hbm.at[pl.ds(row0 * LANE, CR * LANE)], upd)
            for j in range(CR):
                pltpu.sync_copy(upd.at[pl.ds(j * LANE, LANE)],
                                acc_shared.at[idx.at[j]], add=True)
            return carry
        lax.fori_loop(0, S_CHUNKS, chunk, 0)

    @pl.when(cid == 0)
    def _():
        run(outm_hbm)

    @pl.when(cid == 1)
    def _():
        run(outw_hbm)

    plsc.subcore_barrier()

    @pl.when(cid == 0)
    def _():
        pltpu.sync_copy(acc_shared.at[pl.ds(sid * NSTRIPE, NSTRIPE)],
                        accm_hbm.at[pl.ds(sid * NSTRIPE, NSTRIPE)])

    @pl.when(cid == 1)
    def _():
        pltpu.sync_copy(acc_shared.at[pl.ds(sid * NSTRIPE, NSTRIPE)],
                        accw_hbm.at[pl.ds(sid * NSTRIPE, NSTRIPE)])


def _sc_scatter(outm, outw, dstp, zeros_n16):
    mesh = plsc.VectorSubcoreMesh(core_axis_name="c", subcore_axis_name="s")
    f = pl.kernel(
        _sc_scatter_body,
        out_type=(jax.ShapeDtypeStruct((N_NODES, M_DIM), jnp.float32),
                  jax.ShapeDtypeStruct((N_NODES, M_DIM), jnp.float32)),
        mesh=mesh,
        scratch_types=[
            pltpu.VMEM((CR, LANE), jnp.int32),
            pltpu.VMEM((CR * LANE, M_DIM), jnp.float32),
            pltpu.VMEM_SHARED((N_NODES, M_DIM), jnp.float32),
            pltpu.SemaphoreType.DMA,
        ],
        compiler_params=pltpu.CompilerParams(use_tc_tiling_on_sc=False),
    )
    return f(outm, outw, dstp, zeros_n16)


def _tc_edge_body(gd_ref, gs_ref, a1_ref, b1_ref, w1d_ref, b1b_ref,
                  w2_ref, b2_ref, cw1_ref, cb1_ref, cw2_ref, cb2_ref,
                  outm_ref, outw_ref):
    pid = pl.program_id(0)
    gd = gd_ref[...]
    gs = gs_ref[...]
    mask3 = (lax.broadcasted_iota(jnp.int32, (1, XD), 1) < POS
             ).astype(jnp.float32)
    rel = (gs - gd) * mask3
    rel_dist = jnp.sum(rel * rel, axis=1, keepdims=True)
    h1 = (jnp.dot(gd, a1_ref[...], preferred_element_type=jnp.float32)
          + jnp.dot(gs, b1_ref[...], preferred_element_type=jnp.float32)
          + rel_dist * w1d_ref[...] + b1b_ref[...])
    a1 = _silu(h1)
    m = _silu(jnp.dot(a1, w2_ref[...], preferred_element_type=jnp.float32)
              + b2_ref[...])
    c1 = _silu(jnp.dot(m, cw1_ref[...], preferred_element_type=jnp.float32)
               + cb1_ref[...])
    cw = (jnp.dot(c1, cw2_ref[...], preferred_element_type=jnp.float32)
          + cb2_ref[...])
    cw0 = cw[:, 0:1]
    rows = lax.broadcasted_iota(jnp.int32, (BE, 1), 0) + pid * BE
    valid = (rows < N_EDGES).astype(jnp.float32)
    outm_ref[...] = m * valid
    outw_ref[...] = (rel * (cw0 * valid))[:, 0:M_DIM]


def _tc_edge(gd, gs, wts):
    (a1, b1, w1d, b1b, w2, b2, cw1, cb1, cw2, cb2) = wts
    nblk = EP // BE
    full = lambda shape: pl.BlockSpec(shape, lambda i: (0,) * len(shape))
    return pl.pallas_call(
        _tc_edge_body,
        grid=(nblk,),
        in_specs=[
            pl.BlockSpec((BE, XD), lambda i: (i, 0)),
            pl.BlockSpec((BE, XD), lambda i: (i, 0)),
            full((XD, H1)), full((XD, H1)), full((1, H1)), full((1, H1)),
            full((H1, M_DIM)), full((1, M_DIM)),
            full((M_DIM, CW)), full((1, CW)),
            full((CW, 8)), full((1, 8)),
        ],
        out_specs=[
            pl.BlockSpec((BE, M_DIM), lambda i: (i, 0)),
            pl.BlockSpec((BE, M_DIM), lambda i: (i, 0)),
        ],
        out_shape=[
            jax.ShapeDtypeStruct((EP, M_DIM), jnp.float32),
            jax.ShapeDtypeStruct((EP, M_DIM), jnp.float32),
        ],
        compiler_params=pltpu.CompilerParams(
            dimension_semantics=("arbitrary",)),
    )(gd, gs, a1, b1, w1d, b1b, w2, b2, cw1, cb1, cw2, cb2)


def _tc_node_body(x_ref, accm_ref, accw_ref, x1_ref, m1_ref, nb1_ref,
                  nw2_ref, nb2_ref, out_ref):
    x = x_ref[...]
    accm = accm_ref[...]
    accw = accw_ref[...]
    h = _silu(jnp.dot(x, x1_ref[...], preferred_element_type=jnp.float32)
              + jnp.dot(accm, m1_ref[...], preferred_element_type=jnp.float32)
              + nb1_ref[...])
    dx = jnp.dot(h, nw2_ref[...], preferred_element_type=jnp.float32) \
        + nb2_ref[...]
    mask3 = (lax.broadcasted_iota(jnp.int32, (1, XD), 1) < POS
             ).astype(jnp.float32)
    mhat = jnp.concatenate(
        [accw, jnp.zeros((BN, XD - M_DIM), jnp.float32)], axis=1) * mask3
    out_ref[...] = x + dx + mhat


def _tc_node(x, accm, accw, wts):
    (x1, m1, nb1, nw2, nb2) = wts
    nblk = N_NODES // BN
    full = lambda shape: pl.BlockSpec(shape, lambda i: (0,) * len(shape))
    return pl.pallas_call(
        _tc_node_body,
        grid=(nblk,),
        in_specs=[
            pl.BlockSpec((BN, XD), lambda i: (i, 0)),
            pl.BlockSpec((BN, M_DIM), lambda i: (i, 0)),
            pl.BlockSpec((BN, M_DIM), lambda i: (i, 0)),
            full((XD, NH)), full((M_DIM, NH)), full((1, NH)),
            full((NH, XD)), full((1, XD)),
        ],
        out_specs=pl.BlockSpec((BN, XD), lambda i: (i, 0)),
        out_shape=jax.ShapeDtypeStruct((N_NODES, XD), jnp.float32),
        compiler_params=pltpu.CompilerParams(
            dimension_semantics=("arbitrary",)),
    )(x, accm, accw, x1, m1, nb1, nw2, nb2)


def _tc_temb_body(t_ref, fr_ref, w1_ref, b1_ref, w2_ref, b2_ref, out_ref):
    emb5 = t_ref[...] * fr_ref[...]
    emb = jnp.concatenate([jnp.sin(emb5), jnp.cos(emb5)], axis=1)
    h = jnp.dot(emb, w1_ref[...], preferred_element_type=jnp.float32) \
        + b1_ref[...]
    h = 0.5 * h * (1.0 + lax.erf(h / np.float32(np.sqrt(2.0))))
    out_ref[...] = jnp.dot(h, w2_ref[...],
                           preferred_element_type=jnp.float32) + b2_ref[...]


def _tc_temb(t2, w1, b1, w2, b2):
    half = TIME_DIM // 2
    freqs = jnp.asarray(
        np.exp(np.arange(half, dtype=np.float32)
               * (-(np.log(10000.0) / (half - 1)))).reshape(1, half))
    return pl.pallas_call(
        _tc_temb_body,
        out_shape=jax.ShapeDtypeStruct((NUM_GRAPHS, TIME_DIM), jnp.float32),
    )(t2, freqs, w1, b1, w2, b2)


def _tc_prep_body(pos_ref, v_ref, b_ref, temb_ref, out_ref):
    b = b_ref[0, 0, :].reshape(BN, 1)
    gids = lax.broadcasted_iota(jnp.int32, (1, NUM_GRAPHS), 1)
    oh = (b == gids).astype(jnp.float32)
    te = jnp.dot(oh, temb_ref[...], preferred_element_type=jnp.float32)
    base = jnp.concatenate(
        [pos_ref[...], v_ref[...],
         jnp.zeros((BN, XD - POS - ATOM_DIM), jnp.float32)], axis=1)
    out_ref[...] = base + te


def _tc_prep(pos, v, batchp, tembp):
    nblk = N_NODES // BN
    full = lambda shape: pl.BlockSpec(shape, lambda i: (0,) * len(shape))
    return pl.pallas_call(
        _tc_prep_body,
        grid=(nblk,),
        in_specs=[
            pl.BlockSpec((BN, POS), lambda i: (i, 0)),
            pl.BlockSpec((BN, ATOM_DIM), lambda i: (i, 0)),
            pl.BlockSpec((1, 1, BN), lambda i: (i, 0, 0)),
            full((NUM_GRAPHS, XD)),
        ],
        out_specs=pl.BlockSpec((BN, XD), lambda i: (i, 0)),
        out_shape=jax.ShapeDtypeStruct((N_NODES, XD), jnp.float32),
        compiler_params=pltpu.CompilerParams(
            dimension_semantics=("arbitrary",)),
    )(pos, v, batchp, tembp)


def _tc_pool_body(x_ref, b_ref, sh_ref, c24_ref, s_ref):
    pid = pl.program_id(0)
    b = b_ref[0, 0, :].reshape(BN, 1)
    gids = lax.broadcasted_iota(jnp.int32, (1, NUM_GRAPHS), 1)
    oh = (b == gids).astype(jnp.float32)
    y = jnp.dot(x_ref[...], sh_ref[...],
                preferred_element_type=jnp.float32) + c24_ref[...]
    part = lax.dot_general(oh, y, (((0,), (0,)), ((), ())),
                           preferred_element_type=jnp.float32)

    @pl.when(pid == 0)
    def _():
        s_ref[...] = part

    @pl.when(pid != 0)
    def _():
        s_ref[...] = s_ref[...] + part


def _tc_pool(x, batchp, sh, c24):
    nblk = N_NODES // BN
    full = lambda shape: pl.BlockSpec(shape, lambda i: (0,) * len(shape))
    return pl.pallas_call(
        _tc_pool_body,
        grid=(nblk,),
        in_specs=[
            pl.BlockSpec((BN, XD), lambda i: (i, 0)),
            pl.BlockSpec((1, 1, BN), lambda i: (i, 0, 0)),
            full((XD, XD)), full((1, XD)),
        ],
        out_specs=pl.BlockSpec((NUM_GRAPHS, XD), lambda i: (0, 0)),
        out_shape=jax.ShapeDtypeStruct((NUM_GRAPHS, XD), jnp.float32),
        compiler_params=pltpu.CompilerParams(
            dimension_semantics=("arbitrary",)),
    )(x, batchp, sh, c24)


def _tc_head_body(s_ref, w1_ref, b1_ref, w2_ref, b2_ref, out_ref):
    s = s_ref[...]
    cnt = jnp.maximum(s[:, FEATS:FEATS + 1], 1.0)
    p = s / cnt
    h = jnp.maximum(
        jnp.dot(p, w1_ref[...], preferred_element_type=jnp.float32)
        + b1_ref[...], 0.0)
    out_ref[...] = jnp.dot(h, w2_ref[...],
                           preferred_element_type=jnp.float32) + b2_ref[...]


def _tc_head(s, w1, b1, w2, b2):
    return pl.pallas_call(
        _tc_head_body,
        out_shape=jax.ShapeDtypeStruct((NUM_GRAPHS, 8), jnp.float32),
    )(s, w1, b1, w2, b2)


def _pack_layer(p):
    e_w1, e_b1 = p["e_w1"], p["e_b1"]
    e_w2, e_b2 = p["e_w2"], p["e_b2"]
    c_w1, c_b1 = p["c_w1"], p["c_b1"]
    c_w2, c_b2 = p["c_w2"], p["c_b2"]
    n_w1, n_b1 = p["n_w1"], p["n_b1"]
    n_w2, n_b2 = p["n_w2"], p["n_b2"]

    a1 = jnp.zeros((XD, H1), jnp.float32).at[POS:POS + FEATS, :94].set(
        e_w1[:FEATS])
    b1 = jnp.zeros((XD, H1), jnp.float32).at[POS:POS + FEATS, :94].set(
        e_w1[FEATS:2 * FEATS])
    w1d = jnp.zeros((1, H1), jnp.float32).at[0, :94].set(e_w1[2 * FEATS])
    b1b = jnp.zeros((1, H1), jnp.float32).at[0, :94].set(e_b1)
    w2 = jnp.zeros((H1, M_DIM), jnp.float32).at[:94].set(e_w2)
    b2 = e_b2.reshape(1, M_DIM)
    cw1 = c_w1
    cb1 = c_b1.reshape(1, CW)
    cw2 = jnp.zeros((CW, 8), jnp.float32).at[:, 0:1].set(c_w2)
    cb2 = jnp.zeros((1, 8), jnp.float32).at[0, 0].set(c_b2[0])
    ew = (a1, b1, w1d, b1b, w2, b2, cw1, cb1, cw2, cb2)

    x1 = jnp.zeros((XD, NH), jnp.float32).at[POS:POS + FEATS, :46].set(
        n_w1[:FEATS])
    m1 = jnp.zeros((M_DIM, NH), jnp.float32).at[:, :46].set(n_w1[FEATS:])
    nb1 = jnp.zeros((1, NH), jnp.float32).at[0, :46].set(n_b1)
    nw2 = jnp.zeros((NH, XD), jnp.float32).at[:46, POS:POS + FEATS].set(n_w2)
    nb2 = jnp.zeros((1, XD), jnp.float32).at[0, POS:POS + FEATS].set(n_b2)
    nw = (x1, m1, nb1, nw2, nb2)
    return ew, nw


def kernel(ligand_pos, ligand_v, edge_index, t, batch, params):
    npad = EP - N_EDGES
    pad_idx = (jnp.arange(npad, dtype=jnp.int32) * 37) % N_NODES
    srcp = jnp.concatenate([edge_index[0], pad_idx]).reshape(IDX_ROWS, LANE)
    dstp = jnp.concatenate([edge_index[1], pad_idx]).reshape(IDX_ROWS, LANE)
    batchp = batch.reshape(N_NODES // BN, 1, BN)
    zeros_n16 = jnp.zeros((N_NODES, M_DIM), jnp.float32)

    temb = _tc_temb(t.reshape(NUM_GRAPHS, 1),
                    params["te_w1"], params["te_b1"].reshape(1, -1),
                    params["te_w2"], params["te_b2"].reshape(1, -1))
    tembp = jnp.zeros((NUM_GRAPHS, XD), jnp.float32
                      ).at[:, POS + ATOM_DIM:POS + ATOM_DIM + TIME_DIM].set(
                          temb)
    x = _tc_prep(ligand_pos, ligand_v, batchp, tembp)

    for l in range(NUM_LAYERS):
        ew, nw = _pack_layer(params["layers"][l])
        gs, gd = _sc_gather(x, srcp, dstp)
        outm, outw = _tc_edge(gd, gs, ew)
        accm, accw = _sc_scatter(outm, outw, dstp, zeros_n16)
        x = _tc_node(x, accm, accw, nw)

    sh = jnp.zeros((XD, XD), jnp.float32).at[
        POS:POS + FEATS, 0:FEATS].set(jnp.eye(FEATS, dtype=jnp.float32))
    c24 = jnp.zeros((1, XD), jnp.float32).at[0, FEATS].set(1.0)
    s = _tc_pool(x, batchp, sh, c24)
    hw1 = jnp.zeros((XD, NH), jnp.float32).at[:FEATS, :46].set(params["d_w1"])
    hb1 = jnp.zeros((1, NH), jnp.float32).at[0, :46].set(params["d_b1"])
    hw2 = jnp.zeros((NH, 8), jnp.float32).at[:46, :2].set(params["d_w2"])
    hb2 = jnp.zeros((1, 8), jnp.float32).at[0, :2].set(params["d_b2"])
    out8 = _tc_head(s, hw1, hb1, hw2, hb2)
    return out8[:, :2]


# speedup vs baseline: 7.5355x; 1.8659x over previous
import functools

import jax
import jax.numpy as jnp
import numpy as np
from jax import lax
from jax.experimental import pallas as pl
from jax.experimental.pallas import tpu as pltpu
from jax.experimental.pallas import tpu_sc as plsc

N_NODES = 100000
N_EDGES = 1600000
NUM_GRAPHS = 64
ATOM_DIM = 13
TIME_DIM = 10
FEATS = ATOM_DIM + TIME_DIM
POS = 3
M_DIM = 16
NUM_LAYERS = 3

XD = 32
H1 = 96
NH = 48
CW = 64

NC, NS = 2, 16
NW = NC * NS
LANE = 128
EP = 1638400
IDX_ROWS = EP // LANE
ROWS_PER_W = IDX_ROWS // NW
GCR = 5
G_CHUNKS = ROWS_PER_W // GCR
SCR = 5
ROWS_PER_T = IDX_ROWS // NS
S_CHUNKS = ROWS_PER_T // SCR
NSTRIPE = N_NODES // NS

BE = 4096
BN = 2000


def _silu(x):
    return x * jax.nn.sigmoid(x)


def _sc_gather_body(x_hbm, srcp_hbm, dstp_hbm, gs_hbm, gd_hbm,
                    idx_s0, idx_d0, rows_s0, rows_d0,
                    idx_s1, idx_d1, rows_s1, rows_d1,
                    si0, si1, ss0, ss1, sg):
    cid = lax.axis_index("c")
    sid = lax.axis_index("s")
    wid = sid * NC + cid
    bufs = ((idx_s0, idx_d0, rows_s0, rows_d0, si0, ss0),
            (idx_s1, idx_d1, rows_s1, rows_d1, si1, ss1))

    def issue_idx(c, b):
        idx_s, idx_d, _, _, si, _ = bufs[b]
        row0 = wid * ROWS_PER_W + c * GCR
        pltpu.async_copy(srcp_hbm.at[pl.ds(row0, GCR)], idx_s, si)
        pltpu.async_copy(dstp_hbm.at[pl.ds(row0, GCR)], idx_d, si)

    issue_idx(0, 0)
    issue_idx(1, 1)

    def body(i, carry):
        for b in range(2):
            idx_s, idx_d, rows_s, rows_d, si, ss = bufs[b]
            c = 2 * i + b
            @pl.when(c >= 2)
            def _():
                pltpu.make_async_copy(
                    rows_s, gs_hbm.at[pl.ds(0, GCR * LANE)], ss).wait()
                pltpu.make_async_copy(
                    rows_d, gd_hbm.at[pl.ds(0, GCR * LANE)], ss).wait()
            pltpu.make_async_copy(
                srcp_hbm.at[pl.ds(0, GCR)], idx_s, si).wait()
            pltpu.make_async_copy(
                dstp_hbm.at[pl.ds(0, GCR)], idx_d, si).wait()
            cps = []
            for j in range(GCR):
                cps.append(pltpu.async_copy(
                    x_hbm.at[idx_s.at[j]],
                    rows_s.at[pl.ds(j * LANE, LANE)], sg))
                cps.append(pltpu.async_copy(
                    x_hbm.at[idx_d.at[j]],
                    rows_d.at[pl.ds(j * LANE, LANE)], sg))
            for cp in cps:
                cp.wait()
            @pl.when(c + 2 < G_CHUNKS)
            def _():
                issue_idx_dyn(c + 2, b)
            row0 = wid * ROWS_PER_W + c * GCR
            e0 = row0 * LANE
            pltpu.async_copy(rows_s, gs_hbm.at[pl.ds(e0, GCR * LANE)], ss)
            pltpu.async_copy(rows_d, gd_hbm.at[pl.ds(e0, GCR * LANE)], ss)
        return carry

    def issue_idx_dyn(c, b):
        idx_s, idx_d, _, _, si, _ = bufs[b]
        row0 = wid * ROWS_PER_W + c * GCR
        pltpu.async_copy(srcp_hbm.at[pl.ds(row0, GCR)], idx_s, si)
        pltpu.async_copy(dstp_hbm.at[pl.ds(row0, GCR)], idx_d, si)

    lax.fori_loop(0, G_CHUNKS // 2, body, 0)

    for b in range(2):
        _, _, rows_s, rows_d, _, ss = bufs[b]
        pltpu.make_async_copy(
            rows_s, gs_hbm.at[pl.ds(0, GCR * LANE)], ss).wait()
        pltpu.make_async_copy(
            rows_d, gd_hbm.at[pl.ds(0, GCR * LANE)], ss).wait()


def _sc_gather(x, srcp, dstp):
    mesh = plsc.VectorSubcoreMesh(core_axis_name="c", subcore_axis_name="s")
    f = pl.kernel(
        _sc_gather_body,
        out_type=(jax.ShapeDtypeStruct((EP, XD), jnp.float32),
                  jax.ShapeDtypeStruct((EP, XD), jnp.float32)),
        mesh=mesh,
        scratch_types=[
            pltpu.VMEM((GCR, LANE), jnp.int32),
            pltpu.VMEM((GCR, LANE), jnp.int32),
            pltpu.VMEM((GCR * LANE, XD), jnp.float32),
            pltpu.VMEM((GCR * LANE, XD), jnp.float32),
            pltpu.VMEM((GCR, LANE), jnp.int32),
            pltpu.VMEM((GCR, LANE), jnp.int32),
            pltpu.VMEM((GCR * LANE, XD), jnp.float32),
            pltpu.VMEM((GCR * LANE, XD), jnp.float32),
            pltpu.SemaphoreType.DMA,
            pltpu.SemaphoreType.DMA,
            pltpu.SemaphoreType.DMA,
            pltpu.SemaphoreType.DMA,
            pltpu.SemaphoreType.DMA,
        ],
        compiler_params=pltpu.CompilerParams(use_tc_tiling_on_sc=False),
    )
    return f(x, srcp, dstp)


def _sc_scatter_body(out32_hbm, dstp_hbm, zeros_hbm,
                     accm_hbm, accw_hbm, idx0, upd0, idx1, upd1,
                     acc_shared, sl0, sl1, ssc):
    cid = lax.axis_index("c")
    sid = lax.axis_index("s")

    pltpu.sync_copy(zeros_hbm.at[pl.ds(sid * NSTRIPE, NSTRIPE)],
                    acc_shared.at[pl.ds(sid * NSTRIPE, NSTRIPE)])
    plsc.subcore_barrier()

    def run(off):
        bufs = ((idx0, upd0, sl0), (idx1, upd1, sl1))

        def issue_load(c, b):
            idx, upd, sl = bufs[b]
            row0 = sid * ROWS_PER_T + c * SCR
            pltpu.async_copy(dstp_hbm.at[pl.ds(row0, SCR)], idx, sl)
            pltpu.async_copy(
                out32_hbm.at[pl.ds(row0 * LANE, SCR * LANE),
                             pl.ds(off, M_DIM)], upd, sl)

        issue_load(0, 0)
        issue_load(1, 1)

        def body(i, carry):
            for b in range(2):
                idx, upd, sl = bufs[b]
                c = 2 * i + b
                pltpu.make_async_copy(
                    dstp_hbm.at[pl.ds(0, SCR)], idx, sl).wait()
                pltpu.make_async_copy(
                    out32_hbm.at[pl.ds(0, SCR * LANE), pl.ds(off, M_DIM)],
                    upd, sl).wait()
                cps = []
                for j in range(SCR):
                    cps.append(pltpu.async_copy(
                        upd.at[pl.ds(j * LANE, LANE)],
                        acc_shared.at[idx.at[j]], ssc, add=True))
                for cp in cps:
                    cp.wait()
                @pl.when(c + 2 < S_CHUNKS)
                def _():
                    issue_load(c + 2, b)
            return carry

        lax.fori_loop(0, S_CHUNKS // 2, body, 0)

    @pl.when(cid == 0)
    def _():
        run(0)

    @pl.when(cid == 1)
    def _():
        run(M_DIM)

    plsc.subcore_barrier()

    @pl.when(cid == 0)
    def _():
        pltpu.sync_copy(acc_shared.at[pl.ds(sid * NSTRIPE, NSTRIPE)],
                        accm_hbm.at[pl.ds(sid * NSTRIPE, NSTRIPE)])

    @pl.when(cid == 1)
    def _():
        pltpu.sync_copy(acc_shared.at[pl.ds(sid * NSTRIPE, NSTRIPE)],
                        accw_hbm.at[pl.ds(sid * NSTRIPE, NSTRIPE)])


def _sc_scatter(out32, dstp, zeros_n16):
    mesh = plsc.VectorSubcoreMesh(core_axis_name="c", subcore_axis_name="s")
    f = pl.kernel(
        _sc_scatter_body,
        out_type=(jax.ShapeDtypeStruct((N_NODES, M_DIM), jnp.float32),
                  jax.ShapeDtypeStruct((N_NODES, M_DIM), jnp.float32)),
        mesh=mesh,
        scratch_types=[
            pltpu.VMEM((SCR, LANE), jnp.int32),
            pltpu.VMEM((SCR * LANE, M_DIM), jnp.float32),
            pltpu.VMEM((SCR, LANE), jnp.int32),
            pltpu.VMEM((SCR * LANE, M_DIM), jnp.float32),
            pltpu.VMEM_SHARED((N_NODES, M_DIM), jnp.float32),
            pltpu.SemaphoreType.DMA,
            pltpu.SemaphoreType.DMA,
            pltpu.SemaphoreType.DMA,
        ],
        compiler_params=pltpu.CompilerParams(use_tc_tiling_on_sc=False),
    )
    return f(out32, dstp, zeros_n16)


def _tc_edge_body(gd_ref, gs_ref, a1_ref, b1_ref, w1d_ref, b1b_ref,
                  w2_ref, b2_ref, cw1_ref, cb1_ref, cw2_ref, cb2_ref,
                  out_ref):
    pid = pl.program_id(0)
    gd4 = gd_ref[...]
    gs4 = gs_ref[...]
    nb4 = BE // 4
    mask3 = (lax.broadcasted_iota(jnp.int32, (1, XD), 1) < POS
             ).astype(jnp.float32)
    pieces = []
    for g in range(4):
        xd = gd4[:, 32 * g:32 * g + 32]
        xs = gs4[:, 32 * g:32 * g + 32]
        rel = (xs - xd) * mask3
        rel_dist = jnp.sum(rel * rel, axis=1, keepdims=True)
        h1 = (jnp.dot(xd, a1_ref[...], preferred_element_type=jnp.float32)
              + jnp.dot(xs, b1_ref[...], preferred_element_type=jnp.float32)
              + rel_dist * w1d_ref[...] + b1b_ref[...])
        a1 = _silu(h1)
        m = _silu(jnp.dot(a1, w2_ref[...],
                          preferred_element_type=jnp.float32) + b2_ref[...])
        c1 = _silu(jnp.dot(m, cw1_ref[...],
                           preferred_element_type=jnp.float32) + cb1_ref[...])
        cw = (jnp.dot(c1, cw2_ref[...],
                      preferred_element_type=jnp.float32) + cb2_ref[...])
        cw0 = cw[:, 0:1]
        rows = lax.broadcasted_iota(jnp.int32, (nb4, 1), 0) * 4 + g \
            + pid * BE
        valid = (rows < N_EDGES).astype(jnp.float32)
        pieces.append(m * valid)
        pieces.append((rel * (cw0 * valid))[:, 0:M_DIM])
    out_ref[...] = jnp.concatenate(pieces, axis=1)


def _tc_edge(gd, gs, wts):
    (a1, b1, w1d, b1b, w2, b2, cw1, cb1, cw2, cb2) = wts
    nblk = EP // BE
    full = lambda shape: pl.BlockSpec(shape, lambda i: (0,) * len(shape))
    return pl.pallas_call(
        _tc_edge_body,
        grid=(nblk,),
        in_specs=[
            pl.BlockSpec((BE // 4, 128), lambda i: (i, 0)),
            pl.BlockSpec((BE // 4, 128), lambda i: (i, 0)),
            full((XD, H1)), full((XD, H1)), full((1, H1)), full((1, H1)),
            full((H1, M_DIM)), full((1, M_DIM)),
            full((M_DIM, CW)), full((1, CW)),
            full((CW, 8)), full((1, 8)),
        ],
        out_specs=pl.BlockSpec((BE // 4, 128), lambda i: (i, 0)),
        out_shape=jax.ShapeDtypeStruct((EP // 4, 128), jnp.float32),
        compiler_params=pltpu.CompilerParams(
            dimension_semantics=("arbitrary",)),
    )(gd, gs, a1, b1, w1d, b1b, w2, b2, cw1, cb1, cw2, cb2)


def _tc_node_body(x_ref, accm_ref, accw_ref, x1_ref, m1_ref, nb1_ref,
                  nw2_ref, nb2_ref, out_ref):
    x = x_ref[...]
    accm = accm_ref[...]
    accw = accw_ref[...]
    h = _silu(jnp.dot(x, x1_ref[...], preferred_element_type=jnp.float32)
              + jnp.dot(accm, m1_ref[...], preferred_element_type=jnp.float32)
              + nb1_ref[...])
    dx = jnp.dot(h, nw2_ref[...], preferred_element_type=jnp.float32) \
        + nb2_ref[...]
    mask3 = (lax.broadcasted_iota(jnp.int32, (1, XD), 1) < POS
             ).astype(jnp.float32)
    mhat = jnp.concatenate(
        [accw, jnp.zeros((BN, XD - M_DIM), jnp.float32)], axis=1) * mask3
    out_ref[...] = x + dx + mhat


def _tc_node(x, accm, accw, wts):
    (x1, m1, nb1, nw2, nb2) = wts
    nblk = N_NODES // BN
    full = lambda shape: pl.BlockSpec(shape, lambda i: (0,) * len(shape))
    return pl.pallas_call(
        _tc_node_body,
        grid=(nblk,),
        in_specs=[
            pl.BlockSpec((BN, XD), lambda i: (i, 0)),
            pl.BlockSpec((BN, M_DIM), lambda i: (i, 0)),
            pl.BlockSpec((BN, M_DIM), lambda i: (i, 0)),
            full((XD, NH)), full((M_DIM, NH)), full((1, NH)),
            full((NH, XD)), full((1, XD)),
        ],
        out_specs=pl.BlockSpec((BN, XD), lambda i: (i, 0)),
        out_shape=jax.ShapeDtypeStruct((N_NODES, XD), jnp.float32),
        compiler_params=pltpu.CompilerParams(
            dimension_semantics=("arbitrary",)),
    )(x, accm, accw, x1, m1, nb1, nw2, nb2)


def _tc_temb_body(t_ref, fr_ref, w1_ref, b1_ref, w2_ref, b2_ref, out_ref):
    emb5 = t_ref[...] * fr_ref[...]
    emb = jnp.concatenate([jnp.sin(emb5), jnp.cos(emb5)], axis=1)
    h = jnp.dot(emb, w1_ref[...], preferred_element_type=jnp.float32) \
        + b1_ref[...]
    h = 0.5 * h * (1.0 + lax.erf(h / np.float32(np.sqrt(2.0))))
    out_ref[...] = jnp.dot(h, w2_ref[...],
                           preferred_element_type=jnp.float32) + b2_ref[...]


def _tc_temb(t2, w1, b1, w2, b2):
    half = TIME_DIM // 2
    freqs = jnp.asarray(
        np.exp(np.arange(half, dtype=np.float32)
               * (-(np.log(10000.0) / (half - 1)))).reshape(1, half))
    return pl.pallas_call(
        _tc_temb_body,
        out_shape=jax.ShapeDtypeStruct((NUM_GRAPHS, TIME_DIM), jnp.float32),
    )(t2, freqs, w1, b1, w2, b2)


def _tc_prep_body(pos_ref, v_ref, b_ref, temb_ref, out_ref):
    b = b_ref[0, 0, :].reshape(BN, 1)
    gids = lax.broadcasted_iota(jnp.int32, (1, NUM_GRAPHS), 1)
    oh = (b == gids).astype(jnp.float32)
    te = jnp.dot(oh, temb_ref[...], preferred_element_type=jnp.float32)
    base = jnp.concatenate(
        [pos_ref[...], v_ref[...],
         jnp.zeros((BN, XD - POS - ATOM_DIM), jnp.float32)], axis=1)
    out_ref[...] = base + te


def _tc_prep(pos, v, batchp, tembp):
    nblk = N_NODES // BN
    full = lambda shape: pl.BlockSpec(shape, lambda i: (0,) * len(shape))
    return pl.pallas_call(
        _tc_prep_body,
        grid=(nblk,),
        in_specs=[
            pl.BlockSpec((BN, POS), lambda i: (i, 0)),
            pl.BlockSpec((BN, ATOM_DIM), lambda i: (i, 0)),
            pl.BlockSpec((1, 1, BN), lambda i: (i, 0, 0)),
            full((NUM_GRAPHS, XD)),
        ],
        out_specs=pl.BlockSpec((BN, XD), lambda i: (i, 0)),
        out_shape=jax.ShapeDtypeStruct((N_NODES, XD), jnp.float32),
        compiler_params=pltpu.CompilerParams(
            dimension_semantics=("arbitrary",)),
    )(pos, v, batchp, tembp)


def _tc_pool_body(x_ref, b_ref, sh_ref, c24_ref, s_ref):
    pid = pl.program_id(0)
    b = b_ref[0, 0, :].reshape(BN, 1)
    gids = lax.broadcasted_iota(jnp.int32, (1, NUM_GRAPHS), 1)
    oh = (b == gids).astype(jnp.float32)
    y = jnp.dot(x_ref[...], sh_ref[...],
                preferred_element_type=jnp.float32) + c24_ref[...]
    part = lax.dot_general(oh, y, (((0,), (0,)), ((), ())),
                           preferred_element_type=jnp.float32)

    @pl.when(pid == 0)
    def _():
        s_ref[...] = part

    @pl.when(pid != 0)
    def _():
        s_ref[...] = s_ref[...] + part


def _tc_pool(x, batchp, sh, c24):
    nblk = N_NODES // BN
    full = lambda shape: pl.BlockSpec(shape, lambda i: (0,) * len(shape))
    return pl.pallas_call(
        _tc_pool_body,
        grid=(nblk,),
        in_specs=[
            pl.BlockSpec((BN, XD), lambda i: (i, 0)),
            pl.BlockSpec((1, 1, BN), lambda i: (i, 0, 0)),
            full((XD, XD)), full((1, XD)),
        ],
        out_specs=pl.BlockSpec((NUM_GRAPHS, XD), lambda i: (0, 0)),
        out_shape=jax.ShapeDtypeStruct((NUM_GRAPHS, XD), jnp.float32),
        compiler_params=pltpu.CompilerParams(
            dimension_semantics=("arbitrary",)),
    )(x, batchp, sh, c24)


def _tc_head_body(s_ref, w1_ref, b1_ref, w2_ref, b2_ref, out_ref):
    s = s_ref[...]
    cnt = jnp.maximum(s[:, FEATS:FEATS + 1], 1.0)
    p = s / cnt
    h = jnp.maximum(
        jnp.dot(p, w1_ref[...], preferred_element_type=jnp.float32)
        + b1_ref[...], 0.0)
    out_ref[...] = jnp.dot(h, w2_ref[...],
                           preferred_element_type=jnp.float32) + b2_ref[...]


def _tc_head(s, w1, b1, w2, b2):
    return pl.pallas_call(
        _tc_head_body,
        out_shape=jax.ShapeDtypeStruct((NUM_GRAPHS, 8), jnp.float32),
    )(s, w1, b1, w2, b2)


def _pack_layer(p):
    e_w1, e_b1 = p["e_w1"], p["e_b1"]
    e_w2, e_b2 = p["e_w2"], p["e_b2"]
    c_w1, c_b1 = p["c_w1"], p["c_b1"]
    c_w2, c_b2 = p["c_w2"], p["c_b2"]
    n_w1, n_b1 = p["n_w1"], p["n_b1"]
    n_w2, n_b2 = p["n_w2"], p["n_b2"]

    a1 = jnp.zeros((XD, H1), jnp.float32).at[POS:POS + FEATS, :94].set(
        e_w1[:FEATS])
    b1 = jnp.zeros((XD, H1), jnp.float32).at[POS:POS + FEATS, :94].set(
        e_w1[FEATS:2 * FEATS])
    w1d = jnp.zeros((1, H1), jnp.float32).at[0, :94].set(e_w1[2 * FEATS])
    b1b = jnp.zeros((1, H1), jnp.float32).at[0, :94].set(e_b1)
    w2 = jnp.zeros((H1, M_DIM), jnp.float32).at[:94].set(e_w2)
    b2 = e_b2.reshape(1, M_DIM)
    cw1 = c_w1
    cb1 = c_b1.reshape(1, CW)
    cw2 = jnp.zeros((CW, 8), jnp.float32).at[:, 0:1].set(c_w2)
    cb2 = jnp.zeros((1, 8), jnp.float32).at[0, 0].set(c_b2[0])
    ew = (a1, b1, w1d, b1b, w2, b2, cw1, cb1, cw2, cb2)

    x1 = jnp.zeros((XD, NH), jnp.float32).at[POS:POS + FEATS, :46].set(
        n_w1[:FEATS])
    m1 = jnp.zeros((M_DIM, NH), jnp.float32).at[:, :46].set(n_w1[FEATS:])
    nb1 = jnp.zeros((1, NH), jnp.float32).at[0, :46].set(n_b1)
    nw2 = jnp.zeros((NH, XD), jnp.float32).at[:46, POS:POS + FEATS].set(n_w2)
    nb2 = jnp.zeros((1, XD), jnp.float32).at[0, POS:POS + FEATS].set(n_b2)
    nw = (x1, m1, nb1, nw2, nb2)
    return ew, nw


def kernel(ligand_pos, ligand_v, edge_index, t, batch, params):
    npad = EP - N_EDGES
    pad_idx = (jnp.arange(npad, dtype=jnp.int32) * 37) % N_NODES
    srcp = jnp.concatenate([edge_index[0], pad_idx]).reshape(IDX_ROWS, LANE)
    dstp = jnp.concatenate([edge_index[1], pad_idx]).reshape(IDX_ROWS, LANE)
    batchp = batch.reshape(N_NODES // BN, 1, BN)
    zeros_n16 = jnp.zeros((N_NODES, M_DIM), jnp.float32)

    temb = _tc_temb(t.reshape(NUM_GRAPHS, 1),
                    params["te_w1"], params["te_b1"].reshape(1, -1),
                    params["te_w2"], params["te_b2"].reshape(1, -1))
    tembp = jnp.zeros((NUM_GRAPHS, XD), jnp.float32
                      ).at[:, POS + ATOM_DIM:POS + ATOM_DIM + TIME_DIM].set(
                          temb)
    x = _tc_prep(ligand_pos, ligand_v, batchp, tembp)

    for l in range(NUM_LAYERS):
        ew, nw = _pack_layer(params["layers"][l])
        gs, gd = _sc_gather(x, srcp, dstp)
        gs4 = gs.reshape(EP // 4, 128)
        gd4 = gd.reshape(EP // 4, 128)
        out4 = _tc_edge(gd4, gs4, ew)
        accm, accw = _sc_scatter(out4.reshape(EP, XD), dstp, zeros_n16)
        x = _tc_node(x, accm, accw, nw)

    sh = jnp.zeros((XD, XD), jnp.float32).at[
        POS:POS + FEATS, 0:FEATS].set(jnp.eye(FEATS, dtype=jnp.float32))
    c24 = jnp.zeros((1, XD), jnp.float32).at[0, FEATS].set(1.0)
    s = _tc_pool(x, batchp, sh, c24)
    hw1 = jnp.zeros((XD, NH), jnp.float32).at[:FEATS, :46].set(params["d_w1"])
    hb1 = jnp.zeros((1, NH), jnp.float32).at[0, :46].set(params["d_b1"])
    hw2 = jnp.zeros((NH, 8), jnp.float32).at[:46, :2].set(params["d_w2"])
    hb2 = jnp.zeros((1, 8), jnp.float32).at[0, :2].set(params["d_b2"])
    out8 = _tc_head(s, hw1, hb1, hw2, hb2)
    return out8[:, :2]


# speedup vs baseline: 12.3205x; 1.6350x over previous
import functools

import jax
import jax.numpy as jnp
import numpy as np
from jax import lax
from jax.experimental import pallas as pl
from jax.experimental.pallas import tpu as pltpu
from jax.experimental.pallas import tpu_sc as plsc

N_NODES = 100000
N_EDGES = 1600000
NUM_GRAPHS = 64
ATOM_DIM = 13
TIME_DIM = 10
FEATS = ATOM_DIM + TIME_DIM
POS = 3
M_DIM = 16
NUM_LAYERS = 3

XD = 32
H1 = 96
NH = 48
CW = 64

NC, NS = 2, 16
NW = NC * NS
LANE = 128
EP = 1638400
IDX_ROWS = EP // LANE
ROWS_PER_W = IDX_ROWS // NW
GCR = 5
G_CHUNKS = ROWS_PER_W // GCR
SCR = 5
ROWS_PER_T = IDX_ROWS // NS
S_CHUNKS = ROWS_PER_T // SCR
NSTRIPE = N_NODES // NS

BE = 4096
BN = 2000


def _silu(x):
    return x * (0.5 + 0.5 * jnp.tanh(0.5 * x))


def _sc_gather_body(x_hbm, srcp_hbm, dstp_hbm, gs_hbm, gd_hbm,
                    idx_s0, idx_d0, rows_s0, rows_d0,
                    idx_s1, idx_d1, rows_s1, rows_d1,
                    si0, si1, ss0, ss1, sg):
    cid = lax.axis_index("c")
    sid = lax.axis_index("s")
    wid = sid * NC + cid
    bufs = ((idx_s0, idx_d0, rows_s0, rows_d0, si0, ss0),
            (idx_s1, idx_d1, rows_s1, rows_d1, si1, ss1))

    def issue_idx(c, b):
        idx_s, idx_d, _, _, si, _ = bufs[b]
        row0 = wid * ROWS_PER_W + c * GCR
        pltpu.async_copy(srcp_hbm.at[pl.ds(row0, GCR)], idx_s, si)
        pltpu.async_copy(dstp_hbm.at[pl.ds(row0, GCR)], idx_d, si)

    issue_idx(0, 0)
    issue_idx(1, 1)

    def body(i, carry):
        for b in range(2):
            idx_s, idx_d, rows_s, rows_d, si, ss = bufs[b]
            c = 2 * i + b
            @pl.when(c >= 2)
            def _():
                pltpu.make_async_copy(
                    rows_s, gs_hbm.at[pl.ds(0, GCR * LANE)], ss).wait()
                pltpu.make_async_copy(
                    rows_d, gd_hbm.at[pl.ds(0, GCR * LANE)], ss).wait()
            pltpu.make_async_copy(
                srcp_hbm.at[pl.ds(0, GCR)], idx_s, si).wait()
            pltpu.make_async_copy(
                dstp_hbm.at[pl.ds(0, GCR)], idx_d, si).wait()
            cps = []
            for j in range(GCR):
                cps.append(pltpu.async_copy(
                    x_hbm.at[idx_s.at[j]],
                    rows_s.at[pl.ds(j * LANE, LANE)], sg))
                cps.append(pltpu.async_copy(
                    x_hbm.at[idx_d.at[j]],
                    rows_d.at[pl.ds(j * LANE, LANE)], sg))
            for cp in cps:
                cp.wait()
            @pl.when(c + 2 < G_CHUNKS)
            def _():
                issue_idx_dyn(c + 2, b)
            row0 = wid * ROWS_PER_W + c * GCR
            e0 = row0 * LANE
            pltpu.async_copy(rows_s, gs_hbm.at[pl.ds(e0, GCR * LANE)], ss)
            pltpu.async_copy(rows_d, gd_hbm.at[pl.ds(e0, GCR * LANE)], ss)
        return carry

    def issue_idx_dyn(c, b):
        idx_s, idx_d, _, _, si, _ = bufs[b]
        row0 = wid * ROWS_PER_W + c * GCR
        pltpu.async_copy(srcp_hbm.at[pl.ds(row0, GCR)], idx_s, si)
        pltpu.async_copy(dstp_hbm.at[pl.ds(row0, GCR)], idx_d, si)

    lax.fori_loop(0, G_CHUNKS // 2, body, 0)

    for b in range(2):
        _, _, rows_s, rows_d, _, ss = bufs[b]
        pltpu.make_async_copy(
            rows_s, gs_hbm.at[pl.ds(0, GCR * LANE)], ss).wait()
        pltpu.make_async_copy(
            rows_d, gd_hbm.at[pl.ds(0, GCR * LANE)], ss).wait()


def _sc_gather(x, srcp, dstp):
    mesh = plsc.VectorSubcoreMesh(core_axis_name="c", subcore_axis_name="s")
    f = pl.kernel(
        _sc_gather_body,
        out_type=(jax.ShapeDtypeStruct((EP, XD), jnp.float32),
                  jax.ShapeDtypeStruct((EP, XD), jnp.float32)),
        mesh=mesh,
        scratch_types=[
            pltpu.VMEM((GCR, LANE), jnp.int32),
            pltpu.VMEM((GCR, LANE), jnp.int32),
            pltpu.VMEM((GCR * LANE, XD), jnp.float32),
            pltpu.VMEM((GCR * LANE, XD), jnp.float32),
            pltpu.VMEM((GCR, LANE), jnp.int32),
            pltpu.VMEM((GCR, LANE), jnp.int32),
            pltpu.VMEM((GCR * LANE, XD), jnp.float32),
            pltpu.VMEM((GCR * LANE, XD), jnp.float32),
            pltpu.SemaphoreType.DMA,
            pltpu.SemaphoreType.DMA,
            pltpu.SemaphoreType.DMA,
            pltpu.SemaphoreType.DMA,
            pltpu.SemaphoreType.DMA,
        ],
        compiler_params=pltpu.CompilerParams(use_tc_tiling_on_sc=False),
    )
    return f(x, srcp, dstp)


def _sc_scatter_body(out32_hbm, dstp_hbm, zeros_hbm,
                     accm_hbm, accw_hbm, idx0, upd0, idx1, upd1,
                     acc_shared, sl0, sl1, ssc):
    cid = lax.axis_index("c")
    sid = lax.axis_index("s")

    pltpu.sync_copy(zeros_hbm.at[pl.ds(sid * NSTRIPE, NSTRIPE)],
                    acc_shared.at[pl.ds(sid * NSTRIPE, NSTRIPE)])
    plsc.subcore_barrier()

    def run(off):
        bufs = ((idx0, upd0, sl0), (idx1, upd1, sl1))

        def issue_load(c, b):
            idx, upd, sl = bufs[b]
            row0 = sid * ROWS_PER_T + c * SCR
            pltpu.async_copy(dstp_hbm.at[pl.ds(row0, SCR)], idx, sl)
            pltpu.async_copy(
                out32_hbm.at[pl.ds(row0 * LANE, SCR * LANE),
                             pl.ds(off, M_DIM)], upd, sl)

        issue_load(0, 0)
        issue_load(1, 1)

        def body(i, carry):
            for b in range(2):
                idx, upd, sl = bufs[b]
                c = 2 * i + b
                pltpu.make_async_copy(
                    dstp_hbm.at[pl.ds(0, SCR)], idx, sl).wait()
                pltpu.make_async_copy(
                    out32_hbm.at[pl.ds(0, SCR * LANE), pl.ds(off, M_DIM)],
                    upd, sl).wait()
                cps = []
                for j in range(SCR):
                    cps.append(pltpu.async_copy(
                        upd.at[pl.ds(j * LANE, LANE)],
                        acc_shared.at[idx.at[j]], ssc, add=True))
                for cp in cps:
                    cp.wait()
                @pl.when(c + 2 < S_CHUNKS)
                def _():
                    issue_load(c + 2, b)
            return carry

        lax.fori_loop(0, S_CHUNKS // 2, body, 0)

    @pl.when(cid == 0)
    def _():
        run(0)

    @pl.when(cid == 1)
    def _():
        run(M_DIM)

    plsc.subcore_barrier()

    @pl.when(cid == 0)
    def _():
        pltpu.sync_copy(acc_shared.at[pl.ds(sid * NSTRIPE, NSTRIPE)],
                        accm_hbm.at[pl.ds(sid * NSTRIPE, NSTRIPE)])

    @pl.when(cid == 1)
    def _():
        pltpu.sync_copy(acc_shared.at[pl.ds(sid * NSTRIPE, NSTRIPE)],
                        accw_hbm.at[pl.ds(sid * NSTRIPE, NSTRIPE)])


def _sc_scatter(out32, dstp, zeros_n16):
    mesh = plsc.VectorSubcoreMesh(core_axis_name="c", subcore_axis_name="s")
    f = pl.kernel(
        _sc_scatter_body,
        out_type=(jax.ShapeDtypeStruct((N_NODES, M_DIM), jnp.float32),
                  jax.ShapeDtypeStruct((N_NODES, M_DIM), jnp.float32)),
        mesh=mesh,
        scratch_types=[
            pltpu.VMEM((SCR, LANE), jnp.int32),
            pltpu.VMEM((SCR * LANE, M_DIM), jnp.float32),
            pltpu.VMEM((SCR, LANE), jnp.int32),
            pltpu.VMEM((SCR * LANE, M_DIM), jnp.float32),
            pltpu.VMEM_SHARED((N_NODES, M_DIM), jnp.float32),
            pltpu.SemaphoreType.DMA,
            pltpu.SemaphoreType.DMA,
            pltpu.SemaphoreType.DMA,
        ],
        compiler_params=pltpu.CompilerParams(use_tc_tiling_on_sc=False),
    )
    return f(out32, dstp, zeros_n16)


def _tc_edge_body(gd_ref, gs_ref, sel_ref, a1_ref, b1_ref, w1d_ref, b1b_ref,
                  w2_ref, b2_ref, cw1_ref, cb1_ref, cw2_ref, cb2_ref,
                  spread_ref, sh16_ref, out_ref):
    pid = pl.program_id(0)
    gd4 = gd_ref[...]
    gs4 = gs_ref[...]
    nb4 = BE // 4
    lane = lax.broadcasted_iota(jnp.int32, (1, 128), 1)
    mask3 = ((lane % 32) < POS).astype(jnp.float32)
    rel = (gs4 - gd4) * mask3
    rd4 = jnp.dot(rel * rel, sel_ref[...],
                  preferred_element_type=jnp.float32)
    h1 = (jnp.dot(gd4, a1_ref[...], preferred_element_type=jnp.float32)
          + jnp.dot(gs4, b1_ref[...], preferred_element_type=jnp.float32)
          + jnp.dot(rd4, w1d_ref[...], preferred_element_type=jnp.float32)
          + b1b_ref[...])
    a1 = _silu(h1)
    m4 = _silu(jnp.dot(a1, w2_ref[...],
                       preferred_element_type=jnp.float32) + b2_ref[...])
    c1 = _silu(jnp.dot(m4, cw1_ref[...],
                       preferred_element_type=jnp.float32) + cb1_ref[...])
    cw4 = (jnp.dot(c1, cw2_ref[...],
                   preferred_element_type=jnp.float32) + cb2_ref[...])
    cwb = jnp.dot(cw4, spread_ref[...],
                  preferred_element_type=jnp.float32)
    rel_sh = jnp.dot(rel, sh16_ref[...],
                     preferred_element_type=jnp.float32)
    row = lax.broadcasted_iota(jnp.int32, (nb4, 1), 0)
    edge_id = pid * BE + 4 * row + lane // 32
    valid = (edge_id < N_EDGES).astype(jnp.float32)
    out_ref[...] = (m4 + rel_sh * cwb) * valid


def _tc_edge(gd, gs, wts):
    (sel, a1, b1, w1d, b1b, w2, b2, cw1, cb1, cw2, cb2, spread, sh16) = wts
    nblk = EP // BE
    full = lambda shape: pl.BlockSpec(shape, lambda i: (0,) * len(shape))
    return pl.pallas_call(
        _tc_edge_body,
        grid=(nblk,),
        in_specs=[
            pl.BlockSpec((BE // 4, 128), lambda i: (i, 0)),
            pl.BlockSpec((BE // 4, 128), lambda i: (i, 0)),
            full((128, 4)),
            full((128, 384)), full((128, 384)), full((4, 384)),
            full((1, 384)),
            full((384, 128)), full((1, 128)),
            full((128, 256)), full((1, 256)),
            full((256, 128)), full((1, 128)),
            full((128, 128)), full((128, 128)),
        ],
        out_specs=pl.BlockSpec((BE // 4, 128), lambda i: (i, 0)),
        out_shape=jax.ShapeDtypeStruct((EP // 4, 128), jnp.float32),
        compiler_params=pltpu.CompilerParams(
            dimension_semantics=("arbitrary",)),
    )(gd, gs, sel, a1, b1, w1d, b1b, w2, b2, cw1, cb1, cw2, cb2,
      spread, sh16)


def _tc_node_body(x_ref, accm_ref, accw_ref, x1_ref, m1_ref, nb1_ref,
                  nw2_ref, nb2_ref, out_ref):
    x = x_ref[...]
    accm = accm_ref[...]
    accw = accw_ref[...]
    h = _silu(jnp.dot(x, x1_ref[...], preferred_element_type=jnp.float32)
              + jnp.dot(accm, m1_ref[...], preferred_element_type=jnp.float32)
              + nb1_ref[...])
    dx = jnp.dot(h, nw2_ref[...], preferred_element_type=jnp.float32) \
        + nb2_ref[...]
    mask3 = (lax.broadcasted_iota(jnp.int32, (1, XD), 1) < POS
             ).astype(jnp.float32)
    mhat = jnp.concatenate(
        [accw, jnp.zeros((BN, XD - M_DIM), jnp.float32)], axis=1) * mask3
    out_ref[...] = x + dx + mhat


def _tc_node(x, accm, accw, wts):
    (x1, m1, nb1, nw2, nb2) = wts
    nblk = N_NODES // BN
    full = lambda shape: pl.BlockSpec(shape, lambda i: (0,) * len(shape))
    return pl.pallas_call(
        _tc_node_body,
        grid=(nblk,),
        in_specs=[
            pl.BlockSpec((BN, XD), lambda i: (i, 0)),
            pl.BlockSpec((BN, M_DIM), lambda i: (i, 0)),
            pl.BlockSpec((BN, M_DIM), lambda i: (i, 0)),
            full((XD, NH)), full((M_DIM, NH)), full((1, NH)),
            full((NH, XD)), full((1, XD)),
        ],
        out_specs=pl.BlockSpec((BN, XD), lambda i: (i, 0)),
        out_shape=jax.ShapeDtypeStruct((N_NODES, XD), jnp.float32),
        compiler_params=pltpu.CompilerParams(
            dimension_semantics=("arbitrary",)),
    )(x, accm, accw, x1, m1, nb1, nw2, nb2)


def _tc_temb_body(t_ref, fr_ref, w1_ref, b1_ref, w2_ref, b2_ref, out_ref):
    emb5 = t_ref[...] * fr_ref[...]
    emb = jnp.concatenate([jnp.sin(emb5), jnp.cos(emb5)], axis=1)
    h = jnp.dot(emb, w1_ref[...], preferred_element_type=jnp.float32) \
        + b1_ref[...]
    h = 0.5 * h * (1.0 + lax.erf(h / np.float32(np.sqrt(2.0))))
    out_ref[...] = jnp.dot(h, w2_ref[...],
                           preferred_element_type=jnp.float32) + b2_ref[...]


def _tc_temb(t2, w1, b1, w2, b2):
    half = TIME_DIM // 2
    freqs = jnp.asarray(
        np.exp(np.arange(half, dtype=np.float32)
               * (-(np.log(10000.0) / (half - 1)))).reshape(1, half))
    return pl.pallas_call(
        _tc_temb_body,
        out_shape=jax.ShapeDtypeStruct((NUM_GRAPHS, TIME_DIM), jnp.float32),
    )(t2, freqs, w1, b1, w2, b2)


def _tc_prep_body(pos_ref, v_ref, b_ref, temb_ref, out_ref):
    b = b_ref[0, 0, :].reshape(BN, 1)
    gids = lax.broadcasted_iota(jnp.int32, (1, NUM_GRAPHS), 1)
    oh = (b == gids).astype(jnp.float32)
    te = jnp.dot(oh, temb_ref[...], preferred_element_type=jnp.float32)
    base = jnp.concatenate(
        [pos_ref[...], v_ref[...],
         jnp.zeros((BN, XD - POS - ATOM_DIM), jnp.float32)], axis=1)
    out_ref[...] = base + te


def _tc_prep(pos, v, batchp, tembp):
    nblk = N_NODES // BN
    full = lambda shape: pl.BlockSpec(shape, lambda i: (0,) * len(shape))
    return pl.pallas_call(
        _tc_prep_body,
        grid=(nblk,),
        in_specs=[
            pl.BlockSpec((BN, POS), lambda i: (i, 0)),
            pl.BlockSpec((BN, ATOM_DIM), lambda i: (i, 0)),
            pl.BlockSpec((1, 1, BN), lambda i: (i, 0, 0)),
            full((NUM_GRAPHS, XD)),
        ],
        out_specs=pl.BlockSpec((BN, XD), lambda i: (i, 0)),
        out_shape=jax.ShapeDtypeStruct((N_NODES, XD), jnp.float32),
        compiler_params=pltpu.CompilerParams(
            dimension_semantics=("arbitrary",)),
    )(pos, v, batchp, tembp)


def _tc_pool_body(x_ref, b_ref, sh_ref, c24_ref, s_ref):
    pid = pl.program_id(0)
    b = b_ref[0, 0, :].reshape(BN, 1)
    gids = lax.broadcasted_iota(jnp.int32, (1, NUM_GRAPHS), 1)
    oh = (b == gids).astype(jnp.float32)
    y = jnp.dot(x_ref[...], sh_ref[...],
                preferred_element_type=jnp.float32) + c24_ref[...]
    part = lax.dot_general(oh, y, (((0,), (0,)), ((), ())),
                           preferred_element_type=jnp.float32)

    @pl.when(pid == 0)
    def _():
        s_ref[...] = part

    @pl.when(pid != 0)
    def _():
        s_ref[...] = s_ref[...] + part


def _tc_pool(x, batchp, sh, c24):
    nblk = N_NODES // BN
    full = lambda shape: pl.BlockSpec(shape, lambda i: (0,) * len(shape))
    return pl.pallas_call(
        _tc_pool_body,
        grid=(nblk,),
        in_specs=[
            pl.BlockSpec((BN, XD), lambda i: (i, 0)),
            pl.BlockSpec((1, 1, BN), lambda i: (i, 0, 0)),
            full((XD, XD)), full((1, XD)),
        ],
        out_specs=pl.BlockSpec((NUM_GRAPHS, XD), lambda i: (0, 0)),
        out_shape=jax.ShapeDtypeStruct((NUM_GRAPHS, XD), jnp.float32),
        compiler_params=pltpu.CompilerParams(
            dimension_semantics=("arbitrary",)),
    )(x, batchp, sh, c24)


def _tc_head_body(s_ref, w1_ref, b1_ref, w2_ref, b2_ref, out_ref):
    s = s_ref[...]
    cnt = jnp.maximum(s[:, FEATS:FEATS + 1], 1.0)
    p = s / cnt
    h = jnp.maximum(
        jnp.dot(p, w1_ref[...], preferred_element_type=jnp.float32)
        + b1_ref[...], 0.0)
    out_ref[...] = jnp.dot(h, w2_ref[...],
                           preferred_element_type=jnp.float32) + b2_ref[...]


def _tc_head(s, w1, b1, w2, b2):
    return pl.pallas_call(
        _tc_head_body,
        out_shape=jax.ShapeDtypeStruct((NUM_GRAPHS, 8), jnp.float32),
    )(s, w1, b1, w2, b2)


def _pack_layer(p):
    e_w1, e_b1 = p["e_w1"], p["e_b1"]
    e_w2, e_b2 = p["e_w2"], p["e_b2"]
    c_w1, c_b1 = p["c_w1"], p["c_b1"]
    c_w2, c_b2 = p["c_w2"], p["c_b2"]
    n_w1, n_b1 = p["n_w1"], p["n_b1"]
    n_w2, n_b2 = p["n_w2"], p["n_b2"]

    eye4 = jnp.eye(4, dtype=jnp.float32)
    a1 = jnp.zeros((XD, H1), jnp.float32).at[POS:POS + FEATS, :94].set(
        e_w1[:FEATS])
    b1 = jnp.zeros((XD, H1), jnp.float32).at[POS:POS + FEATS, :94].set(
        e_w1[FEATS:2 * FEATS])
    w1d = jnp.zeros((1, H1), jnp.float32).at[0, :94].set(e_w1[2 * FEATS])
    b1b = jnp.zeros((1, H1), jnp.float32).at[0, :94].set(e_b1)
    w2p = jnp.zeros((H1, 32), jnp.float32).at[:94, :M_DIM].set(e_w2)
    b2p = jnp.zeros((1, 32), jnp.float32).at[0, :M_DIM].set(e_b2)
    cw1p = jnp.zeros((32, CW), jnp.float32).at[:M_DIM].set(c_w1)
    cb1 = c_b1.reshape(1, CW)
    cw2p = jnp.zeros((CW, 32), jnp.float32).at[:, M_DIM:M_DIM + 1].set(c_w2)
    cb2p = jnp.zeros((1, 32), jnp.float32).at[0, M_DIM].set(c_b2[0])
    lanes = np.arange(128)
    sel = jnp.asarray(
        (lanes[:, None] // 32 == np.arange(4)[None, :]).astype(np.float32))
    a1_4 = jnp.kron(eye4, a1)
    b1_4 = jnp.kron(eye4, b1)
    w1d_4 = jnp.kron(eye4, w1d)
    b1b_4 = jnp.tile(b1b, (1, 4))
    w2_4 = jnp.kron(eye4, w2p)
    b2_4 = jnp.tile(b2p, (1, 4))
    cw1_4 = jnp.kron(eye4, cw1p)
    cb1_4 = jnp.tile(cb1, (1, 4))
    cw2_4 = jnp.kron(eye4, cw2p)
    cb2_4 = jnp.tile(cb2p, (1, 4))
    spread = np.zeros((128, 128), np.float32)
    sh16 = np.zeros((128, 128), np.float32)
    for g in range(4):
        for k in range(POS):
            spread[32 * g + M_DIM, 32 * g + M_DIM + k] = 1.0
            sh16[32 * g + k, 32 * g + M_DIM + k] = 1.0
    ew = (sel, a1_4, b1_4, w1d_4, b1b_4, w2_4, b2_4, cw1_4, cb1_4,
          cw2_4, cb2_4, jnp.asarray(spread), jnp.asarray(sh16))

    x1 = jnp.zeros((XD, NH), jnp.float32).at[POS:POS + FEATS, :46].set(
        n_w1[:FEATS])
    m1 = jnp.zeros((M_DIM, NH), jnp.float32).at[:, :46].set(n_w1[FEATS:])
    nb1 = jnp.zeros((1, NH), jnp.float32).at[0, :46].set(n_b1)
    nw2 = jnp.zeros((NH, XD), jnp.float32).at[:46, POS:POS + FEATS].set(n_w2)
    nb2 = jnp.zeros((1, XD), jnp.float32).at[0, POS:POS + FEATS].set(n_b2)
    nw = (x1, m1, nb1, nw2, nb2)
    return ew, nw


def kernel(ligand_pos, ligand_v, edge_index, t, batch, params):
    npad = EP - N_EDGES
    pad_idx = (jnp.arange(npad, dtype=jnp.int32) * 37) % N_NODES
    srcp = jnp.concatenate([edge_index[0], pad_idx]).reshape(IDX_ROWS, LANE)
    dstp = jnp.concatenate([edge_index[1], pad_idx]).reshape(IDX_ROWS, LANE)
    batchp = batch.reshape(N_NODES // BN, 1, BN)
    zeros_n16 = jnp.zeros((N_NODES, M_DIM), jnp.float32)

    temb = _tc_temb(t.reshape(NUM_GRAPHS, 1),
                    params["te_w1"], params["te_b1"].reshape(1, -1),
                    params["te_w2"], params["te_b2"].reshape(1, -1))
    tembp = jnp.zeros((NUM_GRAPHS, XD), jnp.float32
                      ).at[:, POS + ATOM_DIM:POS + ATOM_DIM + TIME_DIM].set(
                          temb)
    x = _tc_prep(ligand_pos, ligand_v, batchp, tembp)

    for l in range(NUM_LAYERS):
        ew, nw = _pack_layer(params["layers"][l])
        gs, gd = _sc_gather(x, srcp, dstp)
        gs4 = gs.reshape(EP // 4, 128)
        gd4 = gd.reshape(EP // 4, 128)
        out4 = _tc_edge(gd4, gs4, ew)
        accm, accw = _sc_scatter(out4.reshape(EP, XD), dstp, zeros_n16)
        x = _tc_node(x, accm, accw, nw)

    sh = jnp.zeros((XD, XD), jnp.float32).at[
        POS:POS + FEATS, 0:FEATS].set(jnp.eye(FEATS, dtype=jnp.float32))
    c24 = jnp.zeros((1, XD), jnp.float32).at[0, FEATS].set(1.0)
    s = _tc_pool(x, batchp, sh, c24)
    hw1 = jnp.zeros((XD, NH), jnp.float32).at[:FEATS, :46].set(params["d_w1"])
    hb1 = jnp.zeros((1, NH), jnp.float32).at[0, :46].set(params["d_b1"])
    hw2 = jnp.zeros((NH, 8), jnp.float32).at[:46, :2].set(params["d_w2"])
    hb2 = jnp.zeros((1, 8), jnp.float32).at[0, :2].set(params["d_b2"])
    out8 = _tc_head(s, hw1, hb1, hw2, hb2)
    return out8[:, :2]


# speedup vs baseline: 14.1159x; 1.1457x over previous
import functools

import jax
import jax.numpy as jnp
import numpy as np
from jax import lax
from jax.experimental import pallas as pl
from jax.experimental.pallas import tpu as pltpu
from jax.experimental.pallas import tpu_sc as plsc

N_NODES = 100000
N_EDGES = 1600000
NUM_GRAPHS = 64
ATOM_DIM = 13
TIME_DIM = 10
FEATS = ATOM_DIM + TIME_DIM
POS = 3
M_DIM = 16
NUM_LAYERS = 3

XD = 32
H1 = 96
NH = 48
CW = 64

NC, NS = 2, 16
NW = NC * NS
LANE = 128
EP = 1638400
IDX_ROWS = EP // LANE
EH = EP // 2
IDX_ROWS_H = IDX_ROWS // 2
ROWS_PER_W = IDX_ROWS_H // NW
GCR = 5
G_CHUNKS = ROWS_PER_W // GCR
SCR = 5
ROWS_PER_T = IDX_ROWS_H // NS
S_CHUNKS = ROWS_PER_T // SCR
NSTRIPE = N_NODES // NS

BE = 4096
BN = 2000


def _silu(x):
    return x * (0.5 + 0.5 * jnp.tanh(0.5 * x))


def _sc_gather_body(x_hbm, srcp_hbm, dstp_hbm, gs_hbm, gd_hbm,
                    idx_s0, idx_d0, rows_s0, rows_d0,
                    idx_s1, idx_d1, rows_s1, rows_d1,
                    si0, si1, ss0, ss1, sg):
    cid = lax.axis_index("c")
    sid = lax.axis_index("s")
    wid = sid * NC + cid
    bufs = ((idx_s0, idx_d0, rows_s0, rows_d0, si0, ss0),
            (idx_s1, idx_d1, rows_s1, rows_d1, si1, ss1))

    def issue_idx(c, b):
        idx_s, idx_d, _, _, si, _ = bufs[b]
        row0 = wid * ROWS_PER_W + c * GCR
        pltpu.async_copy(srcp_hbm.at[pl.ds(row0, GCR)], idx_s, si)
        pltpu.async_copy(dstp_hbm.at[pl.ds(row0, GCR)], idx_d, si)

    issue_idx(0, 0)
    issue_idx(1, 1)

    def body(i, carry):
        for b in range(2):
            idx_s, idx_d, rows_s, rows_d, si, ss = bufs[b]
            c = 2 * i + b
            @pl.when(c >= 2)
            def _():
                pltpu.make_async_copy(
                    rows_s, gs_hbm.at[pl.ds(0, GCR * LANE)], ss).wait()
                pltpu.make_async_copy(
                    rows_d, gd_hbm.at[pl.ds(0, GCR * LANE)], ss).wait()
            pltpu.make_async_copy(
                srcp_hbm.at[pl.ds(0, GCR)], idx_s, si).wait()
            pltpu.make_async_copy(
                dstp_hbm.at[pl.ds(0, GCR)], idx_d, si).wait()
            cps = []
            for j in range(GCR):
                cps.append(pltpu.async_copy(
                    x_hbm.at[idx_s.at[j]],
                    rows_s.at[pl.ds(j * LANE, LANE)], sg))
                cps.append(pltpu.async_copy(
                    x_hbm.at[idx_d.at[j]],
                    rows_d.at[pl.ds(j * LANE, LANE)], sg))
            for cp in cps:
                cp.wait()
            @pl.when(c + 2 < G_CHUNKS)
            def _():
                issue_idx_dyn(c + 2, b)
            row0 = wid * ROWS_PER_W + c * GCR
            e0 = row0 * LANE
            pltpu.async_copy(rows_s, gs_hbm.at[pl.ds(e0, GCR * LANE)], ss)
            pltpu.async_copy(rows_d, gd_hbm.at[pl.ds(e0, GCR * LANE)], ss)
        return carry

    def issue_idx_dyn(c, b):
        idx_s, idx_d, _, _, si, _ = bufs[b]
        row0 = wid * ROWS_PER_W + c * GCR
        pltpu.async_copy(srcp_hbm.at[pl.ds(row0, GCR)], idx_s, si)
        pltpu.async_copy(dstp_hbm.at[pl.ds(row0, GCR)], idx_d, si)

    lax.fori_loop(0, G_CHUNKS // 2, body, 0)

    for b in range(2):
        _, _, rows_s, rows_d, _, ss = bufs[b]
        pltpu.make_async_copy(
            rows_s, gs_hbm.at[pl.ds(0, GCR * LANE)], ss).wait()
        pltpu.make_async_copy(
            rows_d, gd_hbm.at[pl.ds(0, GCR * LANE)], ss).wait()


def _sc_gather(x, srcp, dstp):
    mesh = plsc.VectorSubcoreMesh(core_axis_name="c", subcore_axis_name="s")
    f = pl.kernel(
        _sc_gather_body,
        out_type=(jax.ShapeDtypeStruct((EH, XD), jnp.float32),
                  jax.ShapeDtypeStruct((EH, XD), jnp.float32)),
        mesh=mesh,
        scratch_types=[
            pltpu.VMEM((GCR, LANE), jnp.int32),
            pltpu.VMEM((GCR, LANE), jnp.int32),
            pltpu.VMEM((GCR * LANE, XD), jnp.float32),
            pltpu.VMEM((GCR * LANE, XD), jnp.float32),
            pltpu.VMEM((GCR, LANE), jnp.int32),
            pltpu.VMEM((GCR, LANE), jnp.int32),
            pltpu.VMEM((GCR * LANE, XD), jnp.float32),
            pltpu.VMEM((GCR * LANE, XD), jnp.float32),
            pltpu.SemaphoreType.DMA,
            pltpu.SemaphoreType.DMA,
            pltpu.SemaphoreType.DMA,
            pltpu.SemaphoreType.DMA,
            pltpu.SemaphoreType.DMA,
        ],
        compiler_params=pltpu.CompilerParams(use_tc_tiling_on_sc=False),
    )
    return f(x, srcp, dstp)


def _sc_scatter_body(out32_hbm, dstp_hbm, initm_hbm, initw_hbm,
                     accm_hbm, accw_hbm, idx0, upd0, idx1, upd1,
                     acc_shared, sl0, sl1, ssc):
    cid = lax.axis_index("c")
    sid = lax.axis_index("s")

    @pl.when(cid == 0)
    def _():
        pltpu.sync_copy(initm_hbm.at[pl.ds(sid * NSTRIPE, NSTRIPE)],
                        acc_shared.at[pl.ds(sid * NSTRIPE, NSTRIPE)])

    @pl.when(cid == 1)
    def _():
        pltpu.sync_copy(initw_hbm.at[pl.ds(sid * NSTRIPE, NSTRIPE)],
                        acc_shared.at[pl.ds(sid * NSTRIPE, NSTRIPE)])

    plsc.subcore_barrier()

    def run(off):
        bufs = ((idx0, upd0, sl0), (idx1, upd1, sl1))

        def issue_load(c, b):
            idx, upd, sl = bufs[b]
            row0 = sid * ROWS_PER_T + c * SCR
            pltpu.async_copy(dstp_hbm.at[pl.ds(row0, SCR)], idx, sl)
            pltpu.async_copy(
                out32_hbm.at[pl.ds(row0 * LANE, SCR * LANE),
                             pl.ds(off, M_DIM)], upd, sl)

        issue_load(0, 0)
        issue_load(1, 1)

        def body(i, carry):
            for b in range(2):
                idx, upd, sl = bufs[b]
                c = 2 * i + b
                pltpu.make_async_copy(
                    dstp_hbm.at[pl.ds(0, SCR)], idx, sl).wait()
                pltpu.make_async_copy(
                    out32_hbm.at[pl.ds(0, SCR * LANE), pl.ds(off, M_DIM)],
                    upd, sl).wait()
                cps = []
                for j in range(SCR):
                    cps.append(pltpu.async_copy(
                        upd.at[pl.ds(j * LANE, LANE)],
                        acc_shared.at[idx.at[j]], ssc, add=True))
                for cp in cps:
                    cp.wait()
                @pl.when(c + 2 < S_CHUNKS)
                def _():
                    issue_load(c + 2, b)
            return carry

        lax.fori_loop(0, S_CHUNKS // 2, body, 0)

    @pl.when(cid == 0)
    def _():
        run(0)

    @pl.when(cid == 1)
    def _():
        run(M_DIM)

    plsc.subcore_barrier()

    @pl.when(cid == 0)
    def _():
        pltpu.sync_copy(acc_shared.at[pl.ds(sid * NSTRIPE, NSTRIPE)],
                        accm_hbm.at[pl.ds(sid * NSTRIPE, NSTRIPE)])

    @pl.when(cid == 1)
    def _():
        pltpu.sync_copy(acc_shared.at[pl.ds(sid * NSTRIPE, NSTRIPE)],
                        accw_hbm.at[pl.ds(sid * NSTRIPE, NSTRIPE)])


def _sc_scatter(out32, dstp, initm, initw):
    mesh = plsc.VectorSubcoreMesh(core_axis_name="c", subcore_axis_name="s")
    f = pl.kernel(
        _sc_scatter_body,
        out_type=(jax.ShapeDtypeStruct((N_NODES, M_DIM), jnp.float32),
                  jax.ShapeDtypeStruct((N_NODES, M_DIM), jnp.float32)),
        mesh=mesh,
        scratch_types=[
            pltpu.VMEM((SCR, LANE), jnp.int32),
            pltpu.VMEM((SCR * LANE, M_DIM), jnp.float32),
            pltpu.VMEM((SCR, LANE), jnp.int32),
            pltpu.VMEM((SCR * LANE, M_DIM), jnp.float32),
            pltpu.VMEM_SHARED((N_NODES, M_DIM), jnp.float32),
            pltpu.SemaphoreType.DMA,
            pltpu.SemaphoreType.DMA,
            pltpu.SemaphoreType.DMA,
        ],
        compiler_params=pltpu.CompilerParams(use_tc_tiling_on_sc=False),
    )
    return f(out32, dstp, initm, initw)


def _tc_edge_body(base, gd_ref, gs_ref, sel_ref, a1_ref, b1_ref, w1d_ref,
                  b1b_ref, w2_ref, b2_ref, cw1_ref, cb1_ref, cw2_ref,
                  cb2_ref, spread_ref, sh16_ref, out_ref):
    pid = pl.program_id(0)
    gd4 = gd_ref[...]
    gs4 = gs_ref[...]
    nb4 = BE // 4
    lane = lax.broadcasted_iota(jnp.int32, (1, 128), 1)
    mask3 = ((lane % 32) < POS).astype(jnp.float32)
    rel = (gs4 - gd4) * mask3
    rd4 = jnp.dot(rel * rel, sel_ref[...],
                  preferred_element_type=jnp.float32)
    h1 = (jnp.dot(gd4, a1_ref[...], preferred_element_type=jnp.float32)
          + jnp.dot(gs4, b1_ref[...], preferred_element_type=jnp.float32)
          + jnp.dot(rd4, w1d_ref[...], preferred_element_type=jnp.float32)
          + b1b_ref[...])
    a1 = _silu(h1)
    m4 = _silu(jnp.dot(a1, w2_ref[...],
                       preferred_element_type=jnp.float32) + b2_ref[...])
    c1 = _silu(jnp.dot(m4, cw1_ref[...],
                       preferred_element_type=jnp.float32) + cb1_ref[...])
    cw4 = (jnp.dot(c1, cw2_ref[...],
                   preferred_element_type=jnp.float32) + cb2_ref[...])
    cwb = jnp.dot(cw4, spread_ref[...],
                  preferred_element_type=jnp.float32)
    rel_sh = jnp.dot(rel, sh16_ref[...],
                     preferred_element_type=jnp.float32)
    row = lax.broadcasted_iota(jnp.int32, (nb4, 1), 0)
    edge_id = base + pid * BE + 4 * row + lane // 32
    valid = (edge_id < N_EDGES).astype(jnp.float32)
    out_ref[...] = (m4 + rel_sh * cwb) * valid


def _tc_edge(gd, gs, wts, base):
    (sel, a1, b1, w1d, b1b, w2, b2, cw1, cb1, cw2, cb2, spread, sh16) = wts
    nblk = EH // BE
    full = lambda shape: pl.BlockSpec(shape, lambda i: (0,) * len(shape))
    return pl.pallas_call(
        functools.partial(_tc_edge_body, base),
        grid=(nblk,),
        in_specs=[
            pl.BlockSpec((BE // 4, 128), lambda i: (i, 0)),
            pl.BlockSpec((BE // 4, 128), lambda i: (i, 0)),
            full((128, 4)),
            full((128, 384)), full((128, 384)), full((4, 384)),
            full((1, 384)),
            full((384, 128)), full((1, 128)),
            full((128, 256)), full((1, 256)),
            full((256, 128)), full((1, 128)),
            full((128, 128)), full((128, 128)),
        ],
        out_specs=pl.BlockSpec((BE // 4, 128), lambda i: (i, 0)),
        out_shape=jax.ShapeDtypeStruct((EH // 4, 128), jnp.float32),
        compiler_params=pltpu.CompilerParams(
            dimension_semantics=("arbitrary",)),
    )(gd, gs, sel, a1, b1, w1d, b1b, w2, b2, cw1, cb1, cw2, cb2,
      spread, sh16)


def _tc_node_body(x_ref, accm_ref, accw_ref, x1_ref, m1_ref, nb1_ref,
                  nw2_ref, nb2_ref, out_ref):
    x = x_ref[...]
    accm = accm_ref[...]
    accw = accw_ref[...]
    h = _silu(jnp.dot(x, x1_ref[...], preferred_element_type=jnp.float32)
              + jnp.dot(accm, m1_ref[...], preferred_element_type=jnp.float32)
              + nb1_ref[...])
    dx = jnp.dot(h, nw2_ref[...], preferred_element_type=jnp.float32) \
        + nb2_ref[...]
    mask3 = (lax.broadcasted_iota(jnp.int32, (1, XD), 1) < POS
             ).astype(jnp.float32)
    mhat = jnp.concatenate(
        [accw, jnp.zeros((BN, XD - M_DIM), jnp.float32)], axis=1) * mask3
    out_ref[...] = x + dx + mhat


def _tc_node(x, accm, accw, wts):
    (x1, m1, nb1, nw2, nb2) = wts
    nblk = N_NODES // BN
    full = lambda shape: pl.BlockSpec(shape, lambda i: (0,) * len(shape))
    return pl.pallas_call(
        _tc_node_body,
        grid=(nblk,),
        in_specs=[
            pl.BlockSpec((BN, XD), lambda i: (i, 0)),
            pl.BlockSpec((BN, M_DIM), lambda i: (i, 0)),
            pl.BlockSpec((BN, M_DIM), lambda i: (i, 0)),
            full((XD, NH)), full((M_DIM, NH)), full((1, NH)),
            full((NH, XD)), full((1, XD)),
        ],
        out_specs=pl.BlockSpec((BN, XD), lambda i: (i, 0)),
        out_shape=jax.ShapeDtypeStruct((N_NODES, XD), jnp.float32),
        compiler_params=pltpu.CompilerParams(
            dimension_semantics=("arbitrary",)),
    )(x, accm, accw, x1, m1, nb1, nw2, nb2)


def _tc_temb_body(t_ref, fr_ref, w1_ref, b1_ref, w2_ref, b2_ref, out_ref):
    emb5 = t_ref[...] * fr_ref[...]
    emb = jnp.concatenate([jnp.sin(emb5), jnp.cos(emb5)], axis=1)
    h = jnp.dot(emb, w1_ref[...], preferred_element_type=jnp.float32) \
        + b1_ref[...]
    h = 0.5 * h * (1.0 + lax.erf(h / np.float32(np.sqrt(2.0))))
    out_ref[...] = jnp.dot(h, w2_ref[...],
                           preferred_element_type=jnp.float32) + b2_ref[...]


def _tc_temb(t2, w1, b1, w2, b2):
    half = TIME_DIM // 2
    freqs = jnp.asarray(
        np.exp(np.arange(half, dtype=np.float32)
               * (-(np.log(10000.0) / (half - 1)))).reshape(1, half))
    return pl.pallas_call(
        _tc_temb_body,
        out_shape=jax.ShapeDtypeStruct((NUM_GRAPHS, TIME_DIM), jnp.float32),
    )(t2, freqs, w1, b1, w2, b2)


def _tc_prep_body(pos_ref, v_ref, b_ref, temb_ref, out_ref):
    b = b_ref[0, 0, :].reshape(BN, 1)
    gids = lax.broadcasted_iota(jnp.int32, (1, NUM_GRAPHS), 1)
    oh = (b == gids).astype(jnp.float32)
    te = jnp.dot(oh, temb_ref[...], preferred_element_type=jnp.float32)
    base = jnp.concatenate(
        [pos_ref[...], v_ref[...],
         jnp.zeros((BN, XD - POS - ATOM_DIM), jnp.float32)], axis=1)
    out_ref[...] = base + te


def _tc_prep(pos, v, batchp, tembp):
    nblk = N_NODES // BN
    full = lambda shape: pl.BlockSpec(shape, lambda i: (0,) * len(shape))
    return pl.pallas_call(
        _tc_prep_body,
        grid=(nblk,),
        in_specs=[
            pl.BlockSpec((BN, POS), lambda i: (i, 0)),
            pl.BlockSpec((BN, ATOM_DIM), lambda i: (i, 0)),
            pl.BlockSpec((1, 1, BN), lambda i: (i, 0, 0)),
            full((NUM_GRAPHS, XD)),
        ],
        out_specs=pl.BlockSpec((BN, XD), lambda i: (i, 0)),
        out_shape=jax.ShapeDtypeStruct((N_NODES, XD), jnp.float32),
        compiler_params=pltpu.CompilerParams(
            dimension_semantics=("arbitrary",)),
    )(pos, v, batchp, tembp)


def _tc_pool_body(x_ref, b_ref, sh_ref, c24_ref, s_ref):
    pid = pl.program_id(0)
    b = b_ref[0, 0, :].reshape(BN, 1)
    gids = lax.broadcasted_iota(jnp.int32, (1, NUM_GRAPHS), 1)
    oh = (b == gids).astype(jnp.float32)
    y = jnp.dot(x_ref[...], sh_ref[...],
                preferred_element_type=jnp.float32) + c24_ref[...]
    part = lax.dot_general(oh, y, (((0,), (0,)), ((), ())),
                           preferred_element_type=jnp.float32)

    @pl.when(pid == 0)
    def _():
        s_ref[...] = part

    @pl.when(pid != 0)
    def _():
        s_ref[...] = s_ref[...] + part


def _tc_pool(x, batchp, sh, c24):
    nblk = N_NODES // BN
    full = lambda shape: pl.BlockSpec(shape, lambda i: (0,) * len(shape))
    return pl.pallas_call(
        _tc_pool_body,
        grid=(nblk,),
        in_specs=[
            pl.BlockSpec((BN, XD), lambda i: (i, 0)),
            pl.BlockSpec((1, 1, BN), lambda i: (i, 0, 0)),
            full((XD, XD)), full((1, XD)),
        ],
        out_specs=pl.BlockSpec((NUM_GRAPHS, XD), lambda i: (0, 0)),
        out_shape=jax.ShapeDtypeStruct((NUM_GRAPHS, XD), jnp.float32),
        compiler_params=pltpu.CompilerParams(
            dimension_semantics=("arbitrary",)),
    )(x, batchp, sh, c24)


def _tc_head_body(s_ref, w1_ref, b1_ref, w2_ref, b2_ref, out_ref):
    s = s_ref[...]
    cnt = jnp.maximum(s[:, FEATS:FEATS + 1], 1.0)
    p = s / cnt
    h = jnp.maximum(
        jnp.dot(p, w1_ref[...], preferred_element_type=jnp.float32)
        + b1_ref[...], 0.0)
    out_ref[...] = jnp.dot(h, w2_ref[...],
                           preferred_element_type=jnp.float32) + b2_ref[...]


def _tc_head(s, w1, b1, w2, b2):
    return pl.pallas_call(
        _tc_head_body,
        out_shape=jax.ShapeDtypeStruct((NUM_GRAPHS, 8), jnp.float32),
    )(s, w1, b1, w2, b2)


def _pack_layer(p):
    e_w1, e_b1 = p["e_w1"], p["e_b1"]
    e_w2, e_b2 = p["e_w2"], p["e_b2"]
    c_w1, c_b1 = p["c_w1"], p["c_b1"]
    c_w2, c_b2 = p["c_w2"], p["c_b2"]
    n_w1, n_b1 = p["n_w1"], p["n_b1"]
    n_w2, n_b2 = p["n_w2"], p["n_b2"]

    eye4 = jnp.eye(4, dtype=jnp.float32)
    a1 = jnp.zeros((XD, H1), jnp.float32).at[POS:POS + FEATS, :94].set(
        e_w1[:FEATS])
    b1 = jnp.zeros((XD, H1), jnp.float32).at[POS:POS + FEATS, :94].set(
        e_w1[FEATS:2 * FEATS])
    w1d = jnp.zeros((1, H1), jnp.float32).at[0, :94].set(e_w1[2 * FEATS])
    b1b = jnp.zeros((1, H1), jnp.float32).at[0, :94].set(e_b1)
    w2p = jnp.zeros((H1, 32), jnp.float32).at[:94, :M_DIM].set(e_w2)
    b2p = jnp.zeros((1, 32), jnp.float32).at[0, :M_DIM].set(e_b2)
    cw1p = jnp.zeros((32, CW), jnp.float32).at[:M_DIM].set(c_w1)
    cb1 = c_b1.reshape(1, CW)
    cw2p = jnp.zeros((CW, 32), jnp.float32).at[:, M_DIM:M_DIM + 1].set(c_w2)
    cb2p = jnp.zeros((1, 32), jnp.float32).at[0, M_DIM].set(c_b2[0])
    lanes = np.arange(128)
    sel = jnp.asarray(
        (lanes[:, None] // 32 == np.arange(4)[None, :]).astype(np.float32))
    a1_4 = jnp.kron(eye4, a1)
    b1_4 = jnp.kron(eye4, b1)
    w1d_4 = jnp.kron(eye4, w1d)
    b1b_4 = jnp.tile(b1b, (1, 4))
    w2_4 = jnp.kron(eye4, w2p)
    b2_4 = jnp.tile(b2p, (1, 4))
    cw1_4 = jnp.kron(eye4, cw1p)
    cb1_4 = jnp.tile(cb1, (1, 4))
    cw2_4 = jnp.kron(eye4, cw2p)
    cb2_4 = jnp.tile(cb2p, (1, 4))
    spread = np.zeros((128, 128), np.float32)
    sh16 = np.zeros((128, 128), np.float32)
    for g in range(4):
        for k in range(POS):
            spread[32 * g + M_DIM, 32 * g + M_DIM + k] = 1.0
            sh16[32 * g + k, 32 * g + M_DIM + k] = 1.0
    ew = (sel, a1_4, b1_4, w1d_4, b1b_4, w2_4, b2_4, cw1_4, cb1_4,
          cw2_4, cb2_4, jnp.asarray(spread), jnp.asarray(sh16))

    x1 = jnp.zeros((XD, NH), jnp.float32).at[POS:POS + FEATS, :46].set(
        n_w1[:FEATS])
    m1 = jnp.zeros((M_DIM, NH), jnp.float32).at[:, :46].set(n_w1[FEATS:])
    nb1 = jnp.zeros((1, NH), jnp.float32).at[0, :46].set(n_b1)
    nw2 = jnp.zeros((NH, XD), jnp.float32).at[:46, POS:POS + FEATS].set(n_w2)
    nb2 = jnp.zeros((1, XD), jnp.float32).at[0, POS:POS + FEATS].set(n_b2)
    nw = (x1, m1, nb1, nw2, nb2)
    return ew, nw


def kernel(ligand_pos, ligand_v, edge_index, t, batch, params):
    npad = EP - N_EDGES
    pad_idx = (jnp.arange(npad, dtype=jnp.int32) * 37) % N_NODES
    srcp = jnp.concatenate([edge_index[0], pad_idx]).reshape(IDX_ROWS, LANE)
    dstp = jnp.concatenate([edge_index[1], pad_idx]).reshape(IDX_ROWS, LANE)
    batchp = batch.reshape(N_NODES // BN, 1, BN)
    zeros_n16 = jnp.zeros((N_NODES, M_DIM), jnp.float32)

    temb = _tc_temb(t.reshape(NUM_GRAPHS, 1),
                    params["te_w1"], params["te_b1"].reshape(1, -1),
                    params["te_w2"], params["te_b2"].reshape(1, -1))
    tembp = jnp.zeros((NUM_GRAPHS, XD), jnp.float32
                      ).at[:, POS + ATOM_DIM:POS + ATOM_DIM + TIME_DIM].set(
                          temb)
    x = _tc_prep(ligand_pos, ligand_v, batchp, tembp)

    srcp1, srcp2 = srcp[:IDX_ROWS_H], srcp[IDX_ROWS_H:]
    dstp1, dstp2 = dstp[:IDX_ROWS_H], dstp[IDX_ROWS_H:]

    for l in range(NUM_LAYERS):
        ew, nw = _pack_layer(params["layers"][l])
        gs1, gd1 = _sc_gather(x, srcp1, dstp1)
        gs2, gd2 = _sc_gather(x, srcp2, dstp2)
        o1 = _tc_edge(gd1.reshape(EH // 4, 128), gs1.reshape(EH // 4, 128),
                      ew, 0)
        o2 = _tc_edge(gd2.reshape(EH // 4, 128), gs2.reshape(EH // 4, 128),
                      ew, EH)
        am1, aw1 = _sc_scatter(o1.reshape(EH, XD), dstp1,
                               zeros_n16, zeros_n16)
        accm, accw = _sc_scatter(o2.reshape(EH, XD), dstp2, am1, aw1)
        x = _tc_node(x, accm, accw, nw)

    sh = jnp.zeros((XD, XD), jnp.float32).at[
        POS:POS + FEATS, 0:FEATS].set(jnp.eye(FEATS, dtype=jnp.float32))
    c24 = jnp.zeros((1, XD), jnp.float32).at[0, FEATS].set(1.0)
    s = _tc_pool(x, batchp, sh, c24)
    hw1 = jnp.zeros((XD, NH), jnp.float32).at[:FEATS, :46].set(params["d_w1"])
    hb1 = jnp.zeros((1, NH), jnp.float32).at[0, :46].set(params["d_b1"])
    hw2 = jnp.zeros((NH, 8), jnp.float32).at[:46, :2].set(params["d_w2"])
    hb2 = jnp.zeros((1, 8), jnp.float32).at[0, :2].set(params["d_b2"])
    out8 = _tc_head(s, hw1, hb1, hw2, hb2)
    return out8[:, :2]
